# Initial kernel scaffold; baseline (speedup 1.0000x reference)
import jax
import jax.numpy as jnp
from jax.experimental import pallas as pl


def kernel(x, edge_index, edge_attr, batch_idx, Wq1, bq1, Wk1, bk1, Wv1, bv1, We1, Ws1, bs1, Wb1, gn_w, gn_b, gn_ms, Wq2, bq2, Wk2, bk2, Wv2, bv2, We2, Ws2, bs2, Wb2):
    raise NotImplementedError("write your pallas kernel here")


# speedup vs baseline: 30.6898x; 30.6898x over previous
import functools

import jax
import jax.numpy as jnp
import numpy as np
from jax import lax
from jax.experimental import pallas as pl
from jax.experimental.pallas import tpu as pltpu
from jax.experimental.pallas import tpu_sc as plsc

NN = 10000
NE = 320000
NC, NS, NL = 2, 16, 16
NWORK = NC * NS
CB = 128
NCHUNK = NE // CB
NJ_BASE = NCHUNK // NWORK
NJ_EXTRA = NCHUNK - NJ_BASE * NWORK
INIT_TILES = 10
RPT = NN // INIT_TILES
ISQRT5 = float(1.0 / np.sqrt(5.0))
NB = 2000
EB = 16000

_SC_PARAMS = pltpu.CompilerParams(use_tc_tiling_on_sc=False)


def _node1_body(x_ref, wq_ref, bq_ref, wk_ref, bk_ref, wv_ref, bv_ref,
                ws_ref, bs_ref, p1_ref, p1k_ref, p1v_ref,
                q_ref, kv_ref, xr_ref):
    x = x_ref[...]
    q = jnp.dot(x, wq_ref[...], preferred_element_type=jnp.float32) + bq_ref[...]
    k = jnp.dot(x, wk_ref[...], preferred_element_type=jnp.float32) + bk_ref[...]
    v = jnp.dot(x, wv_ref[...], preferred_element_type=jnp.float32) + bv_ref[...]
    q_ref[...] = jnp.dot(q, p1_ref[...], preferred_element_type=jnp.float32)
    kv_ref[...] = (jnp.dot(k, p1k_ref[...], preferred_element_type=jnp.float32)
                   + jnp.dot(v, p1v_ref[...], preferred_element_type=jnp.float32))
    xr_ref[...] = jnp.dot(x, ws_ref[...], preferred_element_type=jnp.float32) + bs_ref[...]


def _eproj_body(ea_ref, w1_ref, w2_ref, p1_ref, p2_ref, e1_ref, e2_ref):
    ea = ea_ref[...]
    e1 = jnp.dot(ea, w1_ref[...], preferred_element_type=jnp.float32)
    e1_ref[...] = jnp.dot(e1, p1_ref[...], preferred_element_type=jnp.float32)
    e2 = jnp.dot(ea, w2_ref[...], preferred_element_type=jnp.float32)
    e2_ref[...] = jnp.dot(e2, p2_ref[...], preferred_element_type=jnp.float32)


def _gate1_body(a0_ref, a1_ref, xr1_ref, snn_ref, dbb_ref, mh_ref,
                wba_ref, wbb_ref, h1_ref):
    t = a0_ref[...] + a1_ref[...]
    num = jnp.dot(t, snn_ref[...], preferred_element_type=jnp.float32)
    den_b = jnp.dot(t, dbb_ref[...], preferred_element_type=jnp.float32) + 1e-16
    out5 = jnp.dot(num / den_b, mh_ref[...], preferred_element_type=jnp.float32)
    xr = xr1_ref[...]
    b = jax.nn.sigmoid(jnp.dot(out5, wba_ref[...], preferred_element_type=jnp.float32)
                       + jnp.dot(xr, wbb_ref[...], preferred_element_type=jnp.float32))
    h1_ref[...] = b * xr + (1.0 - b) * out5


def _gstats_body(h1_ref, bidxT_ref, gnw_ref, gnms_ref, a_ref, s_ref):
    h1 = h1_ref[...]
    gidT = jax.lax.broadcasted_iota(jnp.int32, (64, NN), 0)
    ohT = (bidxT_ref[...] == gidT).astype(jnp.float32)
    cnt = jnp.sum(ohT, axis=1, keepdims=True)
    inv = 1.0 / jnp.maximum(cnt, 1.0)
    mean_g = jnp.dot(ohT, h1, preferred_element_type=jnp.float32) * inv
    m2_g = jnp.dot(ohT, h1 * h1, preferred_element_type=jnp.float32) * inv
    ms = gnms_ref[...]
    var_g = m2_g - mean_g * mean_g * ms * (2.0 - ms)
    a_ref[...] = mean_g * ms
    s_ref[...] = gnw_ref[...] * jax.lax.rsqrt(var_g + 1e-5)


def _node2_body(h1_ref, bidx_ref, a_ref, s_ref, gnb_ref,
                wq2_ref, bq2_ref, wk2_ref, bk2_ref, wv2_ref, bv2_ref,
                ws2_ref, bs2_ref, p2_ref, p2k_ref, p2v_ref,
                q2_ref, kv2_ref, xr2_ref):
    h1 = h1_ref[...]
    gid = jax.lax.broadcasted_iota(jnp.int32, (NB, 64), 1)
    oh = (bidx_ref[...] == gid).astype(jnp.float32)
    a_n = jnp.dot(oh, a_ref[...], preferred_element_type=jnp.float32)
    s_n = jnp.dot(oh, s_ref[...], preferred_element_type=jnp.float32)
    hh = (h1 - a_n) * s_n + gnb_ref[...]
    hh = jnp.maximum(hh, 0.0)
    q2 = jnp.dot(hh, wq2_ref[...], preferred_element_type=jnp.float32) + bq2_ref[...]
    k2 = jnp.dot(hh, wk2_ref[...], preferred_element_type=jnp.float32) + bk2_ref[...]
    v2 = jnp.dot(hh, wv2_ref[...], preferred_element_type=jnp.float32) + bv2_ref[...]
    q2_ref[...] = jnp.dot(q2, p2_ref[...], preferred_element_type=jnp.float32)
    kv2_ref[...] = (jnp.dot(k2, p2k_ref[...], preferred_element_type=jnp.float32)
                    + jnp.dot(v2, p2v_ref[...], preferred_element_type=jnp.float32))
    xr2_ref[...] = jnp.dot(hh, ws2_ref[...], preferred_element_type=jnp.float32) + bs2_ref[...]


def _final_body(a0_ref, a1_ref, xr2_ref, sn_ref, sd_ref, mh_ref, wa_ref, wb_ref,
                out_ref):
    t = a0_ref[...] + a1_ref[...]
    num = jnp.dot(t, sn_ref[...], preferred_element_type=jnp.float32)
    den = jnp.dot(t, sd_ref[...], preferred_element_type=jnp.float32) + 1e-16
    out1 = jnp.dot(num / den, mh_ref[...], preferred_element_type=jnp.float32)
    xr = xr2_ref[...]
    b = jax.nn.sigmoid(out1 * wa_ref[...] + xr * wb_ref[...])
    out_ref[...] = jax.nn.sigmoid(b * xr + (1.0 - b) * out1)


_MESH = plsc.VectorSubcoreMesh(core_axis_name="c", subcore_axis_name="s")


@functools.partial(
    pl.kernel,
    out_type=jax.ShapeDtypeStruct((NC * NN, 96), jnp.float32),
    mesh=_MESH,
    scratch_types=[
        pltpu.VMEM((CB,), jnp.int32),
        pltpu.VMEM((CB,), jnp.int32),
        pltpu.VMEM((CB, 80), jnp.float32),
        pltpu.VMEM((CB, 160), jnp.float32),
        pltpu.VMEM((CB, 80), jnp.float32),
        pltpu.VMEM((CB, 96), jnp.float32),
        pltpu.VMEM_SHARED((NN, 96), jnp.float32),
        pltpu.SemaphoreType.DMA,
        pltpu.SemaphoreType.DMA,
        pltpu.SemaphoreType.DMA,
    ],
    compiler_params=_SC_PARAMS,
)
def _edge1(src_hbm, dst_hbm, q_hbm, kv_hbm, e_hbm, z_hbm, out_hbm,
           src_v, dst_v, qv, kvv, ev, mv, acc, sem1, sem2, sem3):
    cid = lax.axis_index("c")
    sid = lax.axis_index("s")
    g = cid * NS + sid

    @pl.when(sid < INIT_TILES)
    def _init():
        pltpu.sync_copy(z_hbm.at[pl.ds(sid * RPT, RPT)],
                        acc.at[pl.ds(sid * RPT, RPT)])
    plsc.subcore_barrier()

    nj = jnp.where(g < NJ_EXTRA, NJ_BASE + 1, NJ_BASE)

    def chunk_body(j, carry):
        base = pl.multiple_of((g + NWORK * j) * CB, CB)
        pltpu.sync_copy(src_hbm.at[pl.ds(base, CB)], src_v)
        pltpu.sync_copy(dst_hbm.at[pl.ds(base, CB)], dst_v)
        cp1 = pltpu.async_copy(q_hbm.at[dst_v], qv, sem1)
        cp2 = pltpu.async_copy(kv_hbm.at[src_v], kvv, sem2)
        cp3 = pltpu.async_copy(e_hbm.at[pl.ds(base, CB)], ev, sem3)
        cp1.wait()
        cp2.wait()
        cp3.wait()

        def edge_body(ei, carry2):
            ecols = []
            alpha = jnp.zeros((NL,), jnp.float32)
            for ci in range(5):
                qc = qv[ei, pl.ds(ci * NL, NL)]
                kc = kvv[ei, pl.ds(ci * NL, NL)]
                ec = ev[ei, pl.ds(ci * NL, NL)]
                ecols.append(ec)
                alpha = alpha + qc * (kc + ec)
            w = jnp.exp(alpha * ISQRT5)
            for ci in range(5):
                vc = kvv[ei, pl.ds(80 + ci * NL, NL)]
                mv[ei, pl.ds(ci * NL, NL)] = (vc + ecols[ci]) * w
            mv[ei, pl.ds(80, NL)] = w
            return carry2

        lax.fori_loop(0, CB, edge_body, 0)
        pltpu.sync_copy(mv, acc.at[dst_v], add=True)
        return carry

    lax.fori_loop(0, nj, chunk_body, 0)
    plsc.subcore_barrier()

    @pl.when(sid < INIT_TILES)
    def _writeback():
        pltpu.sync_copy(acc.at[pl.ds(sid * RPT, RPT)],
                        out_hbm.at[pl.ds(cid * NN + sid * RPT, RPT)])


@functools.partial(
    pl.kernel,
    out_type=jax.ShapeDtypeStruct((NC * NN, 32), jnp.float32),
    mesh=_MESH,
    scratch_types=[
        pltpu.VMEM((CB,), jnp.int32),
        pltpu.VMEM((CB,), jnp.int32),
        pltpu.VMEM((CB, 16), jnp.float32),
        pltpu.VMEM((CB, 32), jnp.float32),
        pltpu.VMEM((CB, 16), jnp.float32),
        pltpu.VMEM((CB, 32), jnp.float32),
        pltpu.VMEM_SHARED((NN, 32), jnp.float32),
        pltpu.SemaphoreType.DMA,
        pltpu.SemaphoreType.DMA,
        pltpu.SemaphoreType.DMA,
    ],
    compiler_params=_SC_PARAMS,
)
def _edge2(src_hbm, dst_hbm, q_hbm, kv_hbm, e_hbm, z_hbm, out_hbm,
           src_v, dst_v, qv, kvv, ev, mv, acc, sem1, sem2, sem3):
    cid = lax.axis_index("c")
    sid = lax.axis_index("s")
    g = cid * NS + sid

    @pl.when(sid < INIT_TILES)
    def _init():
        pltpu.sync_copy(z_hbm.at[pl.ds(sid * RPT, RPT)],
                        acc.at[pl.ds(sid * RPT, RPT)])
    plsc.subcore_barrier()

    nj = jnp.where(g < NJ_EXTRA, NJ_BASE + 1, NJ_BASE)

    def chunk_body(j, carry):
        base = pl.multiple_of((g + NWORK * j) * CB, CB)
        pltpu.sync_copy(src_hbm.at[pl.ds(base, CB)], src_v)
        pltpu.sync_copy(dst_hbm.at[pl.ds(base, CB)], dst_v)
        cp1 = pltpu.async_copy(q_hbm.at[dst_v], qv, sem1)
        cp2 = pltpu.async_copy(kv_hbm.at[src_v], kvv, sem2)
        cp3 = pltpu.async_copy(e_hbm.at[pl.ds(base, CB)], ev, sem3)
        cp1.wait()
        cp2.wait()
        cp3.wait()

        def edge_body(ei, carry2):
            qc = qv[ei, pl.ds(0, NL)]
            kc = kvv[ei, pl.ds(0, NL)]
            vc = kvv[ei, pl.ds(NL, NL)]
            ec = ev[ei, pl.ds(0, NL)]
            w = jnp.exp(qc * (kc + ec))
            mv[ei, pl.ds(0, NL)] = (vc + ec) * w
            mv[ei, pl.ds(NL, NL)] = w
            return carry2

        lax.fori_loop(0, CB, edge_body, 0)
        pltpu.sync_copy(mv, acc.at[dst_v], add=True)
        return carry

    lax.fori_loop(0, nj, chunk_body, 0)
    plsc.subcore_barrier()

    @pl.when(sid < INIT_TILES)
    def _writeback():
        pltpu.sync_copy(acc.at[pl.ds(sid * RPT, RPT)],
                        out_hbm.at[pl.ds(cid * NN + sid * RPT, RPT)])


def _perm1():
    p = np.zeros((50, 80), np.float32)
    for h in range(10):
        for c in range(5):
            p[h * 5 + c, c * 16 + h] = 1.0
    return p


def _perm2():
    p = np.zeros((10, 16), np.float32)
    for h in range(10):
        p[h, h] = 1.0
    return p


def _row_spec(width):
    return pl.BlockSpec((NB, width), lambda i: (i, 0))


def _bcast_spec(shape):
    return pl.BlockSpec(shape, lambda i: (0, 0))


def kernel(x, edge_index, edge_attr, batch_idx, Wq1, bq1, Wk1, bk1, Wv1, bv1,
           We1, Ws1, bs1, Wb1, gn_w, gn_b, gn_ms, Wq2, bq2, Wk2, bk2, Wv2, bv2,
           We2, Ws2, bs2, Wb2):
    f32 = jnp.float32
    src = edge_index[0]
    dst = edge_index[1]
    ngrid = NN // NB

    p1 = _perm1()
    p1k = np.concatenate([p1, np.zeros_like(p1)], axis=1)
    p1v = np.concatenate([np.zeros_like(p1), p1], axis=1)
    p2 = _perm2()
    p2k = np.concatenate([p2, np.zeros_like(p2)], axis=1)
    p2v = np.concatenate([np.zeros_like(p2), p2], axis=1)

    q1, kv1, xr1 = pl.pallas_call(
        _node1_body,
        grid=(ngrid,),
        in_specs=[_row_spec(128), _bcast_spec((128, 50)), _bcast_spec((1, 50)),
                  _bcast_spec((128, 50)), _bcast_spec((1, 50)),
                  _bcast_spec((128, 50)), _bcast_spec((1, 50)),
                  _bcast_spec((128, 5)), _bcast_spec((1, 5)),
                  _bcast_spec((50, 80)), _bcast_spec((50, 160)),
                  _bcast_spec((50, 160))],
        out_specs=(_row_spec(80), _row_spec(160), _row_spec(5)),
        out_shape=(jax.ShapeDtypeStruct((NN, 80), f32),
                   jax.ShapeDtypeStruct((NN, 160), f32),
                   jax.ShapeDtypeStruct((NN, 5), f32)),
    )(x, Wq1, bq1[None, :], Wk1, bk1[None, :], Wv1, bv1[None, :],
      Ws1, bs1[None, :], jnp.asarray(p1), jnp.asarray(p1k), jnp.asarray(p1v))

    e1, e2 = pl.pallas_call(
        _eproj_body,
        grid=(NE // EB,),
        in_specs=[pl.BlockSpec((EB, 16), lambda i: (i, 0)),
                  _bcast_spec((16, 50)), _bcast_spec((16, 10)),
                  _bcast_spec((50, 80)), _bcast_spec((10, 16))],
        out_specs=(pl.BlockSpec((EB, 80), lambda i: (i, 0)),
                   pl.BlockSpec((EB, 16), lambda i: (i, 0))),
        out_shape=(jax.ShapeDtypeStruct((NE, 80), f32),
                   jax.ShapeDtypeStruct((NE, 16), f32)),
    )(edge_attr, We1, We2, jnp.asarray(p1), jnp.asarray(p2))

    acc1 = _edge1(src, dst, q1, kv1, e1, jnp.zeros((NN, 96), f32))
    a0 = acc1[:NN]
    a1 = acc1[NN:]

    snn = np.zeros((96, 80), np.float32)
    snn[:80, :] = np.eye(80)
    dbb = np.zeros((96, 80), np.float32)
    for h in range(10):
        for c in range(5):
            dbb[80 + h, c * 16 + h] = 1.0
    mh = np.zeros((80, 5), np.float32)
    for h in range(10):
        for c in range(5):
            mh[c * 16 + h, c] = 0.1
    wba = Wb1[0:5] + Wb1[10:15]
    wbb = Wb1[5:10] - Wb1[10:15]
    h1 = pl.pallas_call(
        _gate1_body,
        grid=(ngrid,),
        in_specs=[_row_spec(96), _row_spec(96), _row_spec(5),
                  _bcast_spec((96, 80)), _bcast_spec((96, 80)),
                  _bcast_spec((80, 5)), _bcast_spec((5, 1)),
                  _bcast_spec((5, 1))],
        out_specs=_row_spec(5),
        out_shape=jax.ShapeDtypeStruct((NN, 5), f32),
    )(a0, a1, xr1, jnp.asarray(snn), jnp.asarray(dbb), jnp.asarray(mh),
      wba, wbb)

    ga, gs = pl.pallas_call(
        _gstats_body,
        out_shape=(jax.ShapeDtypeStruct((64, 5), f32),
                   jax.ShapeDtypeStruct((64, 5), f32)),
    )(h1, batch_idx[None, :], gn_w[None, :], gn_ms[None, :])

    q2, kv2, xr2 = pl.pallas_call(
        _node2_body,
        grid=(ngrid,),
        in_specs=[_row_spec(5), _row_spec(1),
                  _bcast_spec((64, 5)), _bcast_spec((64, 5)),
                  _bcast_spec((1, 5)),
                  _bcast_spec((5, 10)), _bcast_spec((1, 10)),
                  _bcast_spec((5, 10)), _bcast_spec((1, 10)),
                  _bcast_spec((5, 10)), _bcast_spec((1, 10)),
                  _bcast_spec((5, 1)), _bcast_spec((1, 1)),
                  _bcast_spec((10, 16)), _bcast_spec((10, 32)),
                  _bcast_spec((10, 32))],
        out_specs=(_row_spec(16), _row_spec(32), _row_spec(1)),
        out_shape=(jax.ShapeDtypeStruct((NN, 16), f32),
                   jax.ShapeDtypeStruct((NN, 32), f32),
                   jax.ShapeDtypeStruct((NN, 1), f32)),
    )(h1, batch_idx[:, None], ga, gs, gn_b[None, :],
      Wq2, bq2[None, :], Wk2, bk2[None, :], Wv2, bv2[None, :],
      Ws2, bs2[None, :], jnp.asarray(p2), jnp.asarray(p2k), jnp.asarray(p2v))

    acc2 = _edge2(src, dst, q2, kv2, e2, jnp.zeros((NN, 32), f32))
    b0 = acc2[:NN]
    b1 = acc2[NN:]

    sn2 = np.zeros((32, 10), np.float32)
    sn2[:10, :] = np.eye(10)
    sd2 = np.zeros((32, 10), np.float32)
    sd2[16:26, :] = np.eye(10)
    mh2 = np.full((10, 1), 0.1, np.float32)
    wa = (Wb2[0] + Wb2[2])[None, :]
    wb = (Wb2[1] - Wb2[2])[None, :]
    out = pl.pallas_call(
        _final_body,
        grid=(ngrid,),
        in_specs=[_row_spec(32), _row_spec(32), _row_spec(1),
                  _bcast_spec((32, 10)), _bcast_spec((32, 10)),
                  _bcast_spec((10, 1)), _bcast_spec((1, 1)),
                  _bcast_spec((1, 1))],
        out_specs=_row_spec(1),
        out_shape=jax.ShapeDtypeStruct((NN, 1), f32),
    )(b0, b1, xr2, jnp.asarray(sn2), jnp.asarray(sd2), jnp.asarray(mh2),
      wa, wb)
    return out


# speedup vs baseline: 48.0625x; 1.5661x over previous
import functools

import jax
import jax.numpy as jnp
import numpy as np
from jax import lax
from jax.experimental import pallas as pl
from jax.experimental.pallas import tpu as pltpu
from jax.experimental.pallas import tpu_sc as plsc

NN = 10000
NE = 320000
NC, NS, NL = 2, 16, 16
NWORK = NC * NS
CB = 128
NCHUNK = NE // CB
NJ_BASE = NCHUNK // NWORK
NJ_EXTRA = NCHUNK - NJ_BASE * NWORK
INIT_TILES = 10
RPT = NN // INIT_TILES
ISQRT5 = float(1.0 / np.sqrt(5.0))
NB = 2000
EB = 16000

_SC_PARAMS = pltpu.CompilerParams(use_tc_tiling_on_sc=False)


def _node1_body(x_ref, wq_ref, bq_ref, wk_ref, bk_ref, wv_ref, bv_ref,
                ws_ref, bs_ref, p1_ref, p1k_ref, p1v_ref,
                q_ref, kv_ref, xr_ref):
    x = x_ref[...]
    q = jnp.dot(x, wq_ref[...], preferred_element_type=jnp.float32) + bq_ref[...]
    k = jnp.dot(x, wk_ref[...], preferred_element_type=jnp.float32) + bk_ref[...]
    v = jnp.dot(x, wv_ref[...], preferred_element_type=jnp.float32) + bv_ref[...]
    q_ref[...] = jnp.dot(q, p1_ref[...], preferred_element_type=jnp.float32)
    kv_ref[...] = (jnp.dot(k, p1k_ref[...], preferred_element_type=jnp.float32)
                   + jnp.dot(v, p1v_ref[...], preferred_element_type=jnp.float32))
    xr_ref[...] = jnp.dot(x, ws_ref[...], preferred_element_type=jnp.float32) + bs_ref[...]


def _eproj_body(ea_ref, w1_ref, w2_ref, p1_ref, p2_ref, e1_ref, e2_ref):
    ea = ea_ref[...]
    e1 = jnp.dot(ea, w1_ref[...], preferred_element_type=jnp.float32)
    e1_ref[...] = jnp.dot(e1, p1_ref[...], preferred_element_type=jnp.float32)
    e2 = jnp.dot(ea, w2_ref[...], preferred_element_type=jnp.float32)
    e2_ref[...] = jnp.dot(e2, p2_ref[...], preferred_element_type=jnp.float32)


def _gate1_body(a0_ref, a1_ref, xr1_ref, snn_ref, dbb_ref, mh_ref,
                wba_ref, wbb_ref, h1_ref):
    t = a0_ref[...] + a1_ref[...]
    num = jnp.dot(t, snn_ref[...], preferred_element_type=jnp.float32)
    den_b = jnp.dot(t, dbb_ref[...], preferred_element_type=jnp.float32) + 1e-16
    out5 = jnp.dot(num / den_b, mh_ref[...], preferred_element_type=jnp.float32)
    xr = xr1_ref[...]
    b = jax.nn.sigmoid(jnp.dot(out5, wba_ref[...], preferred_element_type=jnp.float32)
                       + jnp.dot(xr, wbb_ref[...], preferred_element_type=jnp.float32))
    h1_ref[...] = b * xr + (1.0 - b) * out5


def _gstats_body(h1_ref, bidxT_ref, gnw_ref, gnms_ref, a_ref, s_ref):
    h1 = h1_ref[...]
    gidT = jax.lax.broadcasted_iota(jnp.int32, (64, NN), 0)
    ohT = (bidxT_ref[...] == gidT).astype(jnp.float32)
    cnt = jnp.sum(ohT, axis=1, keepdims=True)
    inv = 1.0 / jnp.maximum(cnt, 1.0)
    mean_g = jnp.dot(ohT, h1, preferred_element_type=jnp.float32) * inv
    m2_g = jnp.dot(ohT, h1 * h1, preferred_element_type=jnp.float32) * inv
    ms = gnms_ref[...]
    var_g = m2_g - mean_g * mean_g * ms * (2.0 - ms)
    a_ref[...] = mean_g * ms
    s_ref[...] = gnw_ref[...] * jax.lax.rsqrt(var_g + 1e-5)


def _node2_body(h1_ref, bidx_ref, a_ref, s_ref, gnb_ref,
                wq2_ref, bq2_ref, wk2_ref, bk2_ref, wv2_ref, bv2_ref,
                ws2_ref, bs2_ref, p2_ref, p2k_ref, p2v_ref,
                q2_ref, kv2_ref, xr2_ref):
    h1 = h1_ref[...]
    gid = jax.lax.broadcasted_iota(jnp.int32, (NB, 64), 1)
    oh = (bidx_ref[...] == gid).astype(jnp.float32)
    a_n = jnp.dot(oh, a_ref[...], preferred_element_type=jnp.float32)
    s_n = jnp.dot(oh, s_ref[...], preferred_element_type=jnp.float32)
    hh = (h1 - a_n) * s_n + gnb_ref[...]
    hh = jnp.maximum(hh, 0.0)
    q2 = jnp.dot(hh, wq2_ref[...], preferred_element_type=jnp.float32) + bq2_ref[...]
    k2 = jnp.dot(hh, wk2_ref[...], preferred_element_type=jnp.float32) + bk2_ref[...]
    v2 = jnp.dot(hh, wv2_ref[...], preferred_element_type=jnp.float32) + bv2_ref[...]
    q2_ref[...] = jnp.dot(q2, p2_ref[...], preferred_element_type=jnp.float32)
    kv2_ref[...] = (jnp.dot(k2, p2k_ref[...], preferred_element_type=jnp.float32)
                    + jnp.dot(v2, p2v_ref[...], preferred_element_type=jnp.float32))
    xr2_ref[...] = jnp.dot(hh, ws2_ref[...], preferred_element_type=jnp.float32) + bs2_ref[...]


def _final_body(a0_ref, a1_ref, xr2_ref, sn_ref, sd_ref, mh_ref, wa_ref, wb_ref,
                out_ref):
    t = a0_ref[...] + a1_ref[...]
    num = jnp.dot(t, sn_ref[...], preferred_element_type=jnp.float32)
    den = jnp.dot(t, sd_ref[...], preferred_element_type=jnp.float32) + 1e-16
    out1 = jnp.dot(num / den, mh_ref[...], preferred_element_type=jnp.float32)
    xr = xr2_ref[...]
    b = jax.nn.sigmoid(out1 * wa_ref[...] + xr * wb_ref[...])
    out_ref[...] = jax.nn.sigmoid(b * xr + (1.0 - b) * out1)


_MESH = plsc.VectorSubcoreMesh(core_axis_name="c", subcore_axis_name="s")


def _make_edge_kernel(cb, qw, kvw, ew, mw, edge_compute):
    nchunk = NE // cb
    nj_base = nchunk // NWORK
    nj_extra = nchunk - nj_base * NWORK

    @functools.partial(
        pl.kernel,
        out_type=jax.ShapeDtypeStruct((NC * NN, mw), jnp.float32),
        mesh=_MESH,
        scratch_types=[
            pltpu.VMEM((2, cb), jnp.int32),
            pltpu.VMEM((2, cb), jnp.int32),
            pltpu.VMEM((cb, qw), jnp.float32),
            pltpu.VMEM((cb, qw), jnp.float32),
            pltpu.VMEM((cb, kvw), jnp.float32),
            pltpu.VMEM((cb, kvw), jnp.float32),
            pltpu.VMEM((cb, ew), jnp.float32),
            pltpu.VMEM((cb, ew), jnp.float32),
            pltpu.VMEM((cb, mw), jnp.float32),
            pltpu.VMEM((cb, mw), jnp.float32),
            pltpu.VMEM_SHARED((NN, mw), jnp.float32),
            pltpu.SemaphoreType.DMA,
            pltpu.SemaphoreType.DMA,
            pltpu.SemaphoreType.DMA,
            pltpu.SemaphoreType.DMA,
        ],
        compiler_params=_SC_PARAMS,
    )
    def _edge(comb_hbm, q_hbm, kv_hbm, e_hbm, z_hbm, out_hbm,
              comb0, comb1, qv0, qv1, kvv0, kvv1, ev0, ev1, mv0, mv1,
              acc, gsem0, gsem1, isem0, isem1):
        cid = lax.axis_index("c")
        sid = lax.axis_index("s")
        g = cid * NS + sid
        combs = (comb0, comb1)
        qvs = (qv0, qv1)
        kvvs = (kvv0, kvv1)
        evs = (ev0, ev1)
        mvs = (mv0, mv1)
        gsems = (gsem0, gsem1)
        isems = (isem0, isem1)

        @pl.when(sid < INIT_TILES)
        def _init():
            pltpu.sync_copy(z_hbm.at[pl.ds(sid * RPT, RPT)],
                            acc.at[pl.ds(sid * RPT, RPT)])
        plsc.subcore_barrier()

        nj = jnp.where(g < nj_extra, nj_base + 1, nj_base)

        def idx_rows(j):
            return pl.ds(2 * (g + NWORK * j), 2)

        def ebase(j):
            return pl.multiple_of((g + NWORK * j) * cb, cb)

        def fire_gathers(j, b):
            pltpu.async_copy(q_hbm.at[combs[b].at[1]], qvs[b], gsems[b])
            pltpu.async_copy(kv_hbm.at[combs[b].at[0]], kvvs[b], gsems[b])
            pltpu.async_copy(e_hbm.at[pl.ds(ebase(j), cb)], evs[b], gsems[b])

        def wait_gathers(j, b):
            pltpu.make_async_copy(q_hbm.at[combs[b].at[1]], qvs[b], gsems[b]).wait()
            pltpu.make_async_copy(kv_hbm.at[combs[b].at[0]], kvvs[b], gsems[b]).wait()
            pltpu.make_async_copy(e_hbm.at[pl.ds(ebase(j), cb)], evs[b], gsems[b]).wait()

        def compute_scatter(b):
            plsc.parallel_loop(0, cb, 1, unroll=4)(
                functools.partial(edge_compute, qvs[b], kvvs[b], evs[b], mvs[b]))
            pltpu.sync_copy(mvs[b], acc.at[combs[b].at[1]], add=True)

        pltpu.sync_copy(comb_hbm.at[idx_rows(0)], comb0)
        fire_gathers(0, 0)
        pltpu.async_copy(comb_hbm.at[idx_rows(1)], comb1, isem1)

        def pair_body(p, carry):
            j0 = 2 * p
            j1 = j0 + 1
            wait_gathers(j0, 0)
            pltpu.make_async_copy(comb_hbm.at[idx_rows(j1)], comb1, isem1).wait()
            fire_gathers(j1, 1)
            compute_scatter(0)

            @pl.when(j0 + 2 < nj)
            def _():
                pltpu.async_copy(comb_hbm.at[idx_rows(j0 + 2)], comb0, isem0)

            wait_gathers(j1, 1)
            compute_scatter(1)

            @pl.when(j0 + 2 < nj)
            def _():
                pltpu.make_async_copy(
                    comb_hbm.at[idx_rows(j0 + 2)], comb0, isem0).wait()
                fire_gathers(j0 + 2, 0)

            @pl.when(j1 + 2 < nj)
            def _():
                pltpu.async_copy(comb_hbm.at[idx_rows(j1 + 2)], comb1, isem1)
            return carry

        lax.fori_loop(0, nj_base // 2, pair_body, 0)

        @pl.when(nj > nj_base)
        def _tail():
            wait_gathers(nj_base, 0)
            compute_scatter(0)

        plsc.subcore_barrier()

        @pl.when(sid < INIT_TILES)
        def _writeback():
            pltpu.sync_copy(acc.at[pl.ds(sid * RPT, RPT)],
                            out_hbm.at[pl.ds(cid * NN + sid * RPT, RPT)])

    return _edge


def _ec1(qv, kvv, ev, mv, ei):
    ecols = []
    alpha = jnp.zeros((NL,), jnp.float32)
    for ci in range(5):
        qc = qv[ei, pl.ds(ci * NL, NL)]
        kc = kvv[ei, pl.ds(ci * NL, NL)]
        ec = ev[ei, pl.ds(ci * NL, NL)]
        ecols.append(ec)
        alpha = alpha + qc * (kc + ec)
    w = jnp.exp(alpha * ISQRT5)
    for ci in range(5):
        vc = kvv[ei, pl.ds(80 + ci * NL, NL)]
        mv[ei, pl.ds(ci * NL, NL)] = (vc + ecols[ci]) * w
    mv[ei, pl.ds(80, NL)] = w


def _ec2(qv, kvv, ev, mv, ei):
    qc = qv[ei, pl.ds(0, NL)]
    kc = kvv[ei, pl.ds(0, NL)]
    vc = kvv[ei, pl.ds(NL, NL)]
    ec = ev[ei, pl.ds(0, NL)]
    w = jnp.exp(qc * (kc + ec))
    mv[ei, pl.ds(0, NL)] = (vc + ec) * w
    mv[ei, pl.ds(NL, NL)] = w


CB1 = 64
CB2 = 128
_edge1 = _make_edge_kernel(CB1, 80, 160, 80, 96, _ec1)
_edge2 = _make_edge_kernel(CB2, 16, 32, 16, 32, _ec2)


def _perm1():
    p = np.zeros((50, 80), np.float32)
    for h in range(10):
        for c in range(5):
            p[h * 5 + c, c * 16 + h] = 1.0
    return p


def _perm2():
    p = np.zeros((10, 16), np.float32)
    for h in range(10):
        p[h, h] = 1.0
    return p


def _row_spec(width):
    return pl.BlockSpec((NB, width), lambda i: (i, 0))


def _bcast_spec(shape):
    return pl.BlockSpec(shape, lambda i: (0, 0))


def kernel(x, edge_index, edge_attr, batch_idx, Wq1, bq1, Wk1, bk1, Wv1, bv1,
           We1, Ws1, bs1, Wb1, gn_w, gn_b, gn_ms, Wq2, bq2, Wk2, bk2, Wv2, bv2,
           We2, Ws2, bs2, Wb2):
    f32 = jnp.float32
    src = edge_index[0]
    dst = edge_index[1]
    comb1 = jnp.stack([src.reshape(NE // CB1, CB1), dst.reshape(NE // CB1, CB1)],
                      axis=1).reshape(2 * (NE // CB1), CB1)
    comb2 = jnp.stack([src.reshape(NE // CB2, CB2), dst.reshape(NE // CB2, CB2)],
                      axis=1).reshape(2 * (NE // CB2), CB2)
    ngrid = NN // NB

    p1 = _perm1()
    p1k = np.concatenate([p1, np.zeros_like(p1)], axis=1)
    p1v = np.concatenate([np.zeros_like(p1), p1], axis=1)
    p2 = _perm2()
    p2k = np.concatenate([p2, np.zeros_like(p2)], axis=1)
    p2v = np.concatenate([np.zeros_like(p2), p2], axis=1)

    q1, kv1, xr1 = pl.pallas_call(
        _node1_body,
        grid=(ngrid,),
        in_specs=[_row_spec(128), _bcast_spec((128, 50)), _bcast_spec((1, 50)),
                  _bcast_spec((128, 50)), _bcast_spec((1, 50)),
                  _bcast_spec((128, 50)), _bcast_spec((1, 50)),
                  _bcast_spec((128, 5)), _bcast_spec((1, 5)),
                  _bcast_spec((50, 80)), _bcast_spec((50, 160)),
                  _bcast_spec((50, 160))],
        out_specs=(_row_spec(80), _row_spec(160), _row_spec(5)),
        out_shape=(jax.ShapeDtypeStruct((NN, 80), f32),
                   jax.ShapeDtypeStruct((NN, 160), f32),
                   jax.ShapeDtypeStruct((NN, 5), f32)),
    )(x, Wq1, bq1[None, :], Wk1, bk1[None, :], Wv1, bv1[None, :],
      Ws1, bs1[None, :], jnp.asarray(p1), jnp.asarray(p1k), jnp.asarray(p1v))

    e1, e2 = pl.pallas_call(
        _eproj_body,
        grid=(NE // EB,),
        in_specs=[pl.BlockSpec((EB, 16), lambda i: (i, 0)),
                  _bcast_spec((16, 50)), _bcast_spec((16, 10)),
                  _bcast_spec((50, 80)), _bcast_spec((10, 16))],
        out_specs=(pl.BlockSpec((EB, 80), lambda i: (i, 0)),
                   pl.BlockSpec((EB, 16), lambda i: (i, 0))),
        out_shape=(jax.ShapeDtypeStruct((NE, 80), f32),
                   jax.ShapeDtypeStruct((NE, 16), f32)),
    )(edge_attr, We1, We2, jnp.asarray(p1), jnp.asarray(p2))

    acc1 = _edge1(comb1, q1, kv1, e1, jnp.zeros((NN, 96), f32))
    a0 = acc1[:NN]
    a1 = acc1[NN:]

    snn = np.zeros((96, 80), np.float32)
    snn[:80, :] = np.eye(80)
    dbb = np.zeros((96, 80), np.float32)
    for h in range(10):
        for c in range(5):
            dbb[80 + h, c * 16 + h] = 1.0
    mh = np.zeros((80, 5), np.float32)
    for h in range(10):
        for c in range(5):
            mh[c * 16 + h, c] = 0.1
    wba = Wb1[0:5] + Wb1[10:15]
    wbb = Wb1[5:10] - Wb1[10:15]
    h1 = pl.pallas_call(
        _gate1_body,
        grid=(ngrid,),
        in_specs=[_row_spec(96), _row_spec(96), _row_spec(5),
                  _bcast_spec((96, 80)), _bcast_spec((96, 80)),
                  _bcast_spec((80, 5)), _bcast_spec((5, 1)),
                  _bcast_spec((5, 1))],
        out_specs=_row_spec(5),
        out_shape=jax.ShapeDtypeStruct((NN, 5), f32),
    )(a0, a1, xr1, jnp.asarray(snn), jnp.asarray(dbb), jnp.asarray(mh),
      wba, wbb)

    ga, gs = pl.pallas_call(
        _gstats_body,
        out_shape=(jax.ShapeDtypeStruct((64, 5), f32),
                   jax.ShapeDtypeStruct((64, 5), f32)),
    )(h1, batch_idx[None, :], gn_w[None, :], gn_ms[None, :])

    q2, kv2, xr2 = pl.pallas_call(
        _node2_body,
        grid=(ngrid,),
        in_specs=[_row_spec(5), _row_spec(1),
                  _bcast_spec((64, 5)), _bcast_spec((64, 5)),
                  _bcast_spec((1, 5)),
                  _bcast_spec((5, 10)), _bcast_spec((1, 10)),
                  _bcast_spec((5, 10)), _bcast_spec((1, 10)),
                  _bcast_spec((5, 10)), _bcast_spec((1, 10)),
                  _bcast_spec((5, 1)), _bcast_spec((1, 1)),
                  _bcast_spec((10, 16)), _bcast_spec((10, 32)),
                  _bcast_spec((10, 32))],
        out_specs=(_row_spec(16), _row_spec(32), _row_spec(1)),
        out_shape=(jax.ShapeDtypeStruct((NN, 16), f32),
                   jax.ShapeDtypeStruct((NN, 32), f32),
                   jax.ShapeDtypeStruct((NN, 1), f32)),
    )(h1, batch_idx[:, None], ga, gs, gn_b[None, :],
      Wq2, bq2[None, :], Wk2, bk2[None, :], Wv2, bv2[None, :],
      Ws2, bs2[None, :], jnp.asarray(p2), jnp.asarray(p2k), jnp.asarray(p2v))

    acc2 = _edge2(comb2, q2, kv2, e2, jnp.zeros((NN, 32), f32))
    b0 = acc2[:NN]
    b1 = acc2[NN:]

    sn2 = np.zeros((32, 10), np.float32)
    sn2[:10, :] = np.eye(10)
    sd2 = np.zeros((32, 10), np.float32)
    sd2[16:26, :] = np.eye(10)
    mh2 = np.full((10, 1), 0.1, np.float32)
    wa = (Wb2[0] + Wb2[2])[None, :]
    wb = (Wb2[1] - Wb2[2])[None, :]
    out = pl.pallas_call(
        _final_body,
        grid=(ngrid,),
        in_specs=[_row_spec(32), _row_spec(32), _row_spec(1),
                  _bcast_spec((32, 10)), _bcast_spec((32, 10)),
                  _bcast_spec((10, 1)), _bcast_spec((1, 1)),
                  _bcast_spec((1, 1))],
        out_specs=_row_spec(1),
        out_shape=jax.ShapeDtypeStruct((NN, 1), f32),
    )(b0, b1, xr2, jnp.asarray(sn2), jnp.asarray(sd2), jnp.asarray(mh2),
      wa, wb)
    return out


# speedup vs baseline: 49.7610x; 1.0353x over previous
import functools

import jax
import jax.numpy as jnp
import numpy as np
from jax import lax
from jax.experimental import pallas as pl
from jax.experimental.pallas import tpu as pltpu
from jax.experimental.pallas import tpu_sc as plsc

NN = 10000
NE = 320000
NC, NS, NL = 2, 16, 16
NWORK = NC * NS
CB = 128
NCHUNK = NE // CB
NJ_BASE = NCHUNK // NWORK
NJ_EXTRA = NCHUNK - NJ_BASE * NWORK
INIT_TILES = 10
RPT = NN // INIT_TILES
ISQRT5 = float(1.0 / np.sqrt(5.0))
NB = 2000
EB = 16000
QW = 128
KVW = 256
EFW = 128

_SC_PARAMS = pltpu.CompilerParams(use_tc_tiling_on_sc=False)


def _node1_body(x_ref, wq_ref, bq_ref, wkv_ref, bkv_ref, ws_ref, bs_ref,
                q_ref, kv_ref, xr_ref):
    x = x_ref[...]
    q_ref[...] = jnp.dot(x, wq_ref[...], preferred_element_type=jnp.float32) + bq_ref[...]
    kv_ref[...] = jnp.dot(x, wkv_ref[...], preferred_element_type=jnp.float32) + bkv_ref[...]
    xr_ref[...] = jnp.dot(x, ws_ref[...], preferred_element_type=jnp.float32) + bs_ref[...]


def _eproj_body(ea_ref, w_ref, e_ref):
    e_ref[...] = jnp.dot(ea_ref[...], w_ref[...],
                         preferred_element_type=jnp.float32)


def _gate1_body(a0_ref, a1_ref, xr1_ref, snn_ref, dbb_ref, mh_ref,
                wba_ref, wbb_ref, h1_ref):
    t = a0_ref[...] + a1_ref[...]
    num = jnp.dot(t, snn_ref[...], preferred_element_type=jnp.float32)
    den_b = jnp.dot(t, dbb_ref[...], preferred_element_type=jnp.float32) + 1e-16
    out5 = jnp.dot(num / den_b, mh_ref[...], preferred_element_type=jnp.float32)
    xr = xr1_ref[...]
    b = jax.nn.sigmoid(jnp.dot(out5, wba_ref[...], preferred_element_type=jnp.float32)
                       + jnp.dot(xr, wbb_ref[...], preferred_element_type=jnp.float32))
    h1_ref[...] = b * xr + (1.0 - b) * out5


def _gstats_body(h1_ref, bidxT_ref, gnw_ref, gnms_ref, a_ref, s_ref):
    h1 = h1_ref[...]
    gidT = jax.lax.broadcasted_iota(jnp.int32, (64, NN), 0)
    ohT = (bidxT_ref[...] == gidT).astype(jnp.float32)
    cnt = jnp.sum(ohT, axis=1, keepdims=True)
    inv = 1.0 / jnp.maximum(cnt, 1.0)
    mean_g = jnp.dot(ohT, h1, preferred_element_type=jnp.float32) * inv
    m2_g = jnp.dot(ohT, h1 * h1, preferred_element_type=jnp.float32) * inv
    ms = gnms_ref[...]
    var_g = m2_g - mean_g * mean_g * ms * (2.0 - ms)
    a_ref[...] = mean_g * ms
    s_ref[...] = gnw_ref[...] * jax.lax.rsqrt(var_g + 1e-5)


def _node2_body(h1_ref, bidx_ref, a_ref, s_ref, gnb_ref,
                wq2_ref, bq2_ref, wk2_ref, bk2_ref, wv2_ref, bv2_ref,
                ws2_ref, bs2_ref, p2_ref, p2k_ref, p2v_ref,
                q2_ref, kv2_ref, xr2_ref):
    h1 = h1_ref[...]
    gid = jax.lax.broadcasted_iota(jnp.int32, (NB, 64), 1)
    oh = (bidx_ref[...] == gid).astype(jnp.float32)
    a_n = jnp.dot(oh, a_ref[...], preferred_element_type=jnp.float32)
    s_n = jnp.dot(oh, s_ref[...], preferred_element_type=jnp.float32)
    hh = (h1 - a_n) * s_n + gnb_ref[...]
    hh = jnp.maximum(hh, 0.0)
    q2 = jnp.dot(hh, wq2_ref[...], preferred_element_type=jnp.float32) + bq2_ref[...]
    k2 = jnp.dot(hh, wk2_ref[...], preferred_element_type=jnp.float32) + bk2_ref[...]
    v2 = jnp.dot(hh, wv2_ref[...], preferred_element_type=jnp.float32) + bv2_ref[...]
    q2_ref[...] = jnp.dot(q2, p2_ref[...], preferred_element_type=jnp.float32)
    kv2_ref[...] = (jnp.dot(k2, p2k_ref[...], preferred_element_type=jnp.float32)
                    + jnp.dot(v2, p2v_ref[...], preferred_element_type=jnp.float32))
    xr2_ref[...] = jnp.dot(hh, ws2_ref[...], preferred_element_type=jnp.float32) + bs2_ref[...]


def _final_body(a0_ref, a1_ref, xr2_ref, sn_ref, sd_ref, mh_ref, wa_ref, wb_ref,
                out_ref):
    t = a0_ref[...] + a1_ref[...]
    num = jnp.dot(t, sn_ref[...], preferred_element_type=jnp.float32)
    den = jnp.dot(t, sd_ref[...], preferred_element_type=jnp.float32) + 1e-16
    out1 = jnp.dot(num / den, mh_ref[...], preferred_element_type=jnp.float32)
    xr = xr2_ref[...]
    b = jax.nn.sigmoid(out1 * wa_ref[...] + xr * wb_ref[...])
    out_ref[...] = jax.nn.sigmoid(b * xr + (1.0 - b) * out1)


_MESH = plsc.VectorSubcoreMesh(core_axis_name="c", subcore_axis_name="s")


def _make_edge_kernel(cb, qw, kvw, ew, eoff, mw, edge_compute):
    nchunk = NE // cb
    nj_base = nchunk // NWORK
    nj_extra = nchunk - nj_base * NWORK

    @functools.partial(
        pl.kernel,
        out_type=jax.ShapeDtypeStruct((NC * NN, mw), jnp.float32),
        mesh=_MESH,
        scratch_types=[
            pltpu.VMEM((2, cb), jnp.int32),
            pltpu.VMEM((2, cb), jnp.int32),
            pltpu.VMEM((cb, qw), jnp.float32),
            pltpu.VMEM((cb, qw), jnp.float32),
            pltpu.VMEM((cb, kvw), jnp.float32),
            pltpu.VMEM((cb, kvw), jnp.float32),
            pltpu.VMEM((cb, ew), jnp.float32),
            pltpu.VMEM((cb, ew), jnp.float32),
            pltpu.VMEM((cb, mw), jnp.float32),
            pltpu.VMEM((cb, mw), jnp.float32),
            pltpu.VMEM_SHARED((NN, mw), jnp.float32),
            pltpu.SemaphoreType.DMA,
            pltpu.SemaphoreType.DMA,
            pltpu.SemaphoreType.DMA,
            pltpu.SemaphoreType.DMA,
        ],
        compiler_params=_SC_PARAMS,
    )
    def _edge(comb_hbm, q_hbm, kv_hbm, e_hbm, z_hbm, out_hbm,
              comb0, comb1, qv0, qv1, kvv0, kvv1, ev0, ev1, mv0, mv1,
              acc, gsem0, gsem1, isem0, isem1):
        cid = lax.axis_index("c")
        sid = lax.axis_index("s")
        g = cid * NS + sid
        combs = (comb0, comb1)
        qvs = (qv0, qv1)
        kvvs = (kvv0, kvv1)
        evs = (ev0, ev1)
        mvs = (mv0, mv1)
        gsems = (gsem0, gsem1)
        isems = (isem0, isem1)

        @pl.when(sid < INIT_TILES)
        def _init():
            pltpu.sync_copy(z_hbm.at[pl.ds(sid * RPT, RPT)],
                            acc.at[pl.ds(sid * RPT, RPT)])
        plsc.subcore_barrier()

        nj = jnp.where(g < nj_extra, nj_base + 1, nj_base)

        def idx_rows(j):
            return pl.ds(2 * (g + NWORK * j), 2)

        def ebase(j):
            return pl.multiple_of((g + NWORK * j) * cb, cb)

        def fire_gathers(j, b):
            pltpu.async_copy(q_hbm.at[combs[b].at[1]], qvs[b], gsems[b])
            pltpu.async_copy(kv_hbm.at[combs[b].at[0]], kvvs[b], gsems[b])
            pltpu.async_copy(e_hbm.at[pl.ds(ebase(j), cb), pl.ds(eoff, ew)],
                             evs[b], gsems[b])

        def wait_gathers(j, b):
            pltpu.make_async_copy(q_hbm.at[combs[b].at[1]], qvs[b], gsems[b]).wait()
            pltpu.make_async_copy(kv_hbm.at[combs[b].at[0]], kvvs[b], gsems[b]).wait()
            pltpu.make_async_copy(e_hbm.at[pl.ds(ebase(j), cb), pl.ds(eoff, ew)],
                                  evs[b], gsems[b]).wait()

        def compute_scatter(b):
            plsc.parallel_loop(0, cb, 1, unroll=4)(
                functools.partial(edge_compute, qvs[b], kvvs[b], evs[b], mvs[b]))
            pltpu.sync_copy(mvs[b], acc.at[combs[b].at[1]], add=True)

        pltpu.sync_copy(comb_hbm.at[idx_rows(0)], comb0)
        fire_gathers(0, 0)
        pltpu.async_copy(comb_hbm.at[idx_rows(1)], comb1, isem1)

        def pair_body(p, carry):
            j0 = 2 * p
            j1 = j0 + 1
            wait_gathers(j0, 0)
            pltpu.make_async_copy(comb_hbm.at[idx_rows(j1)], comb1, isem1).wait()
            fire_gathers(j1, 1)
            compute_scatter(0)

            @pl.when(j0 + 2 < nj)
            def _():
                pltpu.async_copy(comb_hbm.at[idx_rows(j0 + 2)], comb0, isem0)

            wait_gathers(j1, 1)
            compute_scatter(1)

            @pl.when(j0 + 2 < nj)
            def _():
                pltpu.make_async_copy(
                    comb_hbm.at[idx_rows(j0 + 2)], comb0, isem0).wait()
                fire_gathers(j0 + 2, 0)

            @pl.when(j1 + 2 < nj)
            def _():
                pltpu.async_copy(comb_hbm.at[idx_rows(j1 + 2)], comb1, isem1)
            return carry

        lax.fori_loop(0, nj_base // 2, pair_body, 0)

        @pl.when(nj > nj_base)
        def _tail():
            wait_gathers(nj_base, 0)
            compute_scatter(0)

        plsc.subcore_barrier()

        @pl.when(sid < INIT_TILES)
        def _writeback():
            pltpu.sync_copy(acc.at[pl.ds(sid * RPT, RPT)],
                            out_hbm.at[pl.ds(cid * NN + sid * RPT, RPT)])

    return _edge


def _ec1(qv, kvv, ev, mv, ei):
    ecols = []
    alpha = jnp.zeros((NL,), jnp.float32)
    for ci in range(5):
        qc = qv[ei, pl.ds(ci * NL, NL)]
        kc = kvv[ei, pl.ds(ci * NL, NL)]
        ec = ev[ei, pl.ds(ci * NL, NL)]
        ecols.append(ec)
        alpha = alpha + qc * (kc + ec)
    w = jnp.exp(alpha * ISQRT5)
    for ci in range(5):
        vc = kvv[ei, pl.ds(128 + ci * NL, NL)]
        mv[ei, pl.ds(ci * NL, NL)] = (vc + ecols[ci]) * w
    mv[ei, pl.ds(80, NL)] = w


def _ec2(qv, kvv, ev, mv, ei):
    qc = qv[ei, pl.ds(0, NL)]
    kc = kvv[ei, pl.ds(0, NL)]
    vc = kvv[ei, pl.ds(NL, NL)]
    ec = ev[ei, pl.ds(0, NL)]
    w = jnp.exp(qc * (kc + ec))
    mv[ei, pl.ds(0, NL)] = (vc + ec) * w
    mv[ei, pl.ds(NL, NL)] = w


CB1 = 32
CB2 = 128
_edge1 = _make_edge_kernel(CB1, QW, KVW, 80, 0, 96, _ec1)
_edge2 = _make_edge_kernel(CB2, 16, 32, 16, 80, 32, _ec2)


def _perm1():
    p = np.zeros((50, 80), np.float32)
    for h in range(10):
        for c in range(5):
            p[h * 5 + c, c * 16 + h] = 1.0
    return p


def _perm2():
    p = np.zeros((10, 16), np.float32)
    for h in range(10):
        p[h, h] = 1.0
    return p


def _row_spec(width):
    return pl.BlockSpec((NB, width), lambda i: (i, 0))


def _bcast_spec(shape):
    return pl.BlockSpec(shape, lambda i: (0, 0))


def kernel(x, edge_index, edge_attr, batch_idx, Wq1, bq1, Wk1, bk1, Wv1, bv1,
           We1, Ws1, bs1, Wb1, gn_w, gn_b, gn_ms, Wq2, bq2, Wk2, bk2, Wv2, bv2,
           We2, Ws2, bs2, Wb2):
    f32 = jnp.float32
    src = edge_index[0]
    dst = edge_index[1]
    comb1 = jnp.stack([src.reshape(NE // CB1, CB1), dst.reshape(NE // CB1, CB1)],
                      axis=1).reshape(2 * (NE // CB1), CB1)
    comb2 = jnp.stack([src.reshape(NE // CB2, CB2), dst.reshape(NE // CB2, CB2)],
                      axis=1).reshape(2 * (NE // CB2), CB2)
    ngrid = NN // NB

    p1 = _perm1()
    p1k = np.concatenate([p1, np.zeros_like(p1)], axis=1)
    p1v = np.concatenate([np.zeros_like(p1), p1], axis=1)
    p2 = _perm2()
    p2k = np.concatenate([p2, np.zeros_like(p2)], axis=1)
    p2v = np.concatenate([np.zeros_like(p2), p2], axis=1)

    p1_128 = np.zeros((50, QW), np.float32)
    p1_128[:, :80] = p1
    p1k256 = np.zeros((50, KVW), np.float32)
    p1k256[:, :80] = p1
    p1v256 = np.zeros((50, KVW), np.float32)
    p1v256[:, 128:208] = p1
    wq_pad = Wq1 @ p1_128
    bq_pad = (bq1 @ p1_128)[None, :]
    wkv_pad = Wk1 @ p1k256 + Wv1 @ p1v256
    bkv_pad = (bk1 @ p1k256 + bv1 @ p1v256)[None, :]
    wef = jnp.concatenate([We1 @ p1, We2 @ p2,
                           jnp.zeros((16, EFW - 96), f32)], axis=1)

    q1, kv1, xr1 = pl.pallas_call(
        _node1_body,
        grid=(ngrid,),
        in_specs=[_row_spec(128), _bcast_spec((128, QW)), _bcast_spec((1, QW)),
                  _bcast_spec((128, KVW)), _bcast_spec((1, KVW)),
                  _bcast_spec((128, 5)), _bcast_spec((1, 5))],
        out_specs=(_row_spec(QW), _row_spec(KVW), _row_spec(5)),
        out_shape=(jax.ShapeDtypeStruct((NN, QW), f32),
                   jax.ShapeDtypeStruct((NN, KVW), f32),
                   jax.ShapeDtypeStruct((NN, 5), f32)),
    )(x, wq_pad, bq_pad, wkv_pad, bkv_pad, Ws1, bs1[None, :])

    ef = pl.pallas_call(
        _eproj_body,
        grid=(NE // EB,),
        in_specs=[pl.BlockSpec((EB, 16), lambda i: (i, 0)),
                  _bcast_spec((16, EFW))],
        out_specs=pl.BlockSpec((EB, EFW), lambda i: (i, 0)),
        out_shape=jax.ShapeDtypeStruct((NE, EFW), f32),
    )(edge_attr, wef)

    acc1 = _edge1(comb1, q1, kv1, ef, jnp.zeros((NN, 96), f32))
    a0 = acc1[:NN]
    a1 = acc1[NN:]

    snn = np.zeros((96, 80), np.float32)
    snn[:80, :] = np.eye(80)
    dbb = np.zeros((96, 80), np.float32)
    for h in range(10):
        for c in range(5):
            dbb[80 + h, c * 16 + h] = 1.0
    mh = np.zeros((80, 5), np.float32)
    for h in range(10):
        for c in range(5):
            mh[c * 16 + h, c] = 0.1
    wba = Wb1[0:5] + Wb1[10:15]
    wbb = Wb1[5:10] - Wb1[10:15]
    h1 = pl.pallas_call(
        _gate1_body,
        grid=(ngrid,),
        in_specs=[_row_spec(96), _row_spec(96), _row_spec(5),
                  _bcast_spec((96, 80)), _bcast_spec((96, 80)),
                  _bcast_spec((80, 5)), _bcast_spec((5, 1)),
                  _bcast_spec((5, 1))],
        out_specs=_row_spec(5),
        out_shape=jax.ShapeDtypeStruct((NN, 5), f32),
    )(a0, a1, xr1, jnp.asarray(snn), jnp.asarray(dbb), jnp.asarray(mh),
      wba, wbb)

    ga, gs = pl.pallas_call(
        _gstats_body,
        out_shape=(jax.ShapeDtypeStruct((64, 5), f32),
                   jax.ShapeDtypeStruct((64, 5), f32)),
    )(h1, batch_idx[None, :], gn_w[None, :], gn_ms[None, :])

    q2, kv2, xr2 = pl.pallas_call(
        _node2_body,
        grid=(ngrid,),
        in_specs=[_row_spec(5), _row_spec(1),
                  _bcast_spec((64, 5)), _bcast_spec((64, 5)),
                  _bcast_spec((1, 5)),
                  _bcast_spec((5, 10)), _bcast_spec((1, 10)),
                  _bcast_spec((5, 10)), _bcast_spec((1, 10)),
                  _bcast_spec((5, 10)), _bcast_spec((1, 10)),
                  _bcast_spec((5, 1)), _bcast_spec((1, 1)),
                  _bcast_spec((10, 16)), _bcast_spec((10, 32)),
                  _bcast_spec((10, 32))],
        out_specs=(_row_spec(16), _row_spec(32), _row_spec(1)),
        out_shape=(jax.ShapeDtypeStruct((NN, 16), f32),
                   jax.ShapeDtypeStruct((NN, 32), f32),
                   jax.ShapeDtypeStruct((NN, 1), f32)),
    )(h1, batch_idx[:, None], ga, gs, gn_b[None, :],
      Wq2, bq2[None, :], Wk2, bk2[None, :], Wv2, bv2[None, :],
      Ws2, bs2[None, :], jnp.asarray(p2), jnp.asarray(p2k), jnp.asarray(p2v))

    acc2 = _edge2(comb2, q2, kv2, ef, jnp.zeros((NN, 32), f32))
    b0 = acc2[:NN]
    b1 = acc2[NN:]

    sn2 = np.zeros((32, 10), np.float32)
    sn2[:10, :] = np.eye(10)
    sd2 = np.zeros((32, 10), np.float32)
    sd2[16:26, :] = np.eye(10)
    mh2 = np.full((10, 1), 0.1, np.float32)
    wa = (Wb2[0] + Wb2[2])[None, :]
    wb = (Wb2[1] - Wb2[2])[None, :]
    out = pl.pallas_call(
        _final_body,
        grid=(ngrid,),
        in_specs=[_row_spec(32), _row_spec(32), _row_spec(1),
                  _bcast_spec((32, 10)), _bcast_spec((32, 10)),
                  _bcast_spec((10, 1)), _bcast_spec((1, 1)),
                  _bcast_spec((1, 1))],
        out_specs=_row_spec(1),
        out_shape=jax.ShapeDtypeStruct((NN, 1), f32),
    )(b0, b1, xr2, jnp.asarray(sn2), jnp.asarray(sd2), jnp.asarray(mh2),
      wa, wb)
    return out


# speedup vs baseline: 74.8185x; 1.5036x over previous
import functools

import jax
import jax.numpy as jnp
import numpy as np
from jax import lax
from jax.experimental import pallas as pl
from jax.experimental.pallas import tpu as pltpu
from jax.experimental.pallas import tpu_sc as plsc

NN = 10000
NE = 320000
NC, NS, NL = 2, 16, 16
NWORK = NC * NS
CB = 128
NCHUNK = NE // CB
NJ_BASE = NCHUNK // NWORK
NJ_EXTRA = NCHUNK - NJ_BASE * NWORK
INIT_TILES = 10
RPT = NN // INIT_TILES
ISQRT5 = float(1.0 / np.sqrt(5.0))
NB = 2000
EB = 16000
QW = 80
KVW = 160
EFW = 128

_SC_PARAMS = pltpu.CompilerParams(use_tc_tiling_on_sc=False)


def _node1_body(x_ref, wq_ref, bq_ref, wkv_ref, bkv_ref, ws_ref, bs_ref,
                q_ref, kv_ref, xr_ref):
    x = x_ref[...]
    q_ref[...] = jnp.dot(x, wq_ref[...], preferred_element_type=jnp.float32) + bq_ref[...]
    kv_ref[...] = jnp.dot(x, wkv_ref[...], preferred_element_type=jnp.float32) + bkv_ref[...]
    xr_ref[...] = jnp.dot(x, ws_ref[...], preferred_element_type=jnp.float32) + bs_ref[...]


def _eproj_body(eat_ref, w_ref, e_ref):
    e_ref[...] = jax.lax.dot_general(
        eat_ref[...], w_ref[...], (((0,), (0,)), ((), ())),
        preferred_element_type=jnp.float32)


def _gate1_body(a0_ref, a1_ref, xr1_ref, snn_ref, dbb_ref, mh_ref,
                wba_ref, wbb_ref, h1_ref):
    t = a0_ref[...] + a1_ref[...]
    num = jnp.dot(t, snn_ref[...], preferred_element_type=jnp.float32)
    den_b = jnp.dot(t, dbb_ref[...], preferred_element_type=jnp.float32) + 1e-16
    out5 = jnp.dot(num / den_b, mh_ref[...], preferred_element_type=jnp.float32)
    xr = xr1_ref[...]
    b = jax.nn.sigmoid(jnp.dot(out5, wba_ref[...], preferred_element_type=jnp.float32)
                       + jnp.dot(xr, wbb_ref[...], preferred_element_type=jnp.float32))
    h1_ref[...] = b * xr + (1.0 - b) * out5


def _gstats_body(h1_ref, bidxT_ref, gnw_ref, gnms_ref, a_ref, s_ref):
    h1 = h1_ref[...]
    gidT = jax.lax.broadcasted_iota(jnp.int32, (64, NN), 0)
    ohT = (bidxT_ref[...] == gidT).astype(jnp.float32)
    cnt = jnp.sum(ohT, axis=1, keepdims=True)
    inv = 1.0 / jnp.maximum(cnt, 1.0)
    mean_g = jnp.dot(ohT, h1, preferred_element_type=jnp.float32) * inv
    m2_g = jnp.dot(ohT, h1 * h1, preferred_element_type=jnp.float32) * inv
    ms = gnms_ref[...]
    var_g = m2_g - mean_g * mean_g * ms * (2.0 - ms)
    a_ref[...] = mean_g * ms
    s_ref[...] = gnw_ref[...] * jax.lax.rsqrt(var_g + 1e-5)


def _node2_body(h1_ref, bidx_ref, a_ref, s_ref, gnb_ref,
                wq2_ref, bq2_ref, wk2_ref, bk2_ref, wv2_ref, bv2_ref,
                ws2_ref, bs2_ref, p2_ref, p2k_ref, p2v_ref,
                q2_ref, kv2_ref, xr2_ref):
    h1 = h1_ref[...]
    gid = jax.lax.broadcasted_iota(jnp.int32, (NB, 64), 1)
    oh = (bidx_ref[...] == gid).astype(jnp.float32)
    a_n = jnp.dot(oh, a_ref[...], preferred_element_type=jnp.float32)
    s_n = jnp.dot(oh, s_ref[...], preferred_element_type=jnp.float32)
    hh = (h1 - a_n) * s_n + gnb_ref[...]
    hh = jnp.maximum(hh, 0.0)
    q2 = jnp.dot(hh, wq2_ref[...], preferred_element_type=jnp.float32) + bq2_ref[...]
    k2 = jnp.dot(hh, wk2_ref[...], preferred_element_type=jnp.float32) + bk2_ref[...]
    v2 = jnp.dot(hh, wv2_ref[...], preferred_element_type=jnp.float32) + bv2_ref[...]
    q2_ref[...] = jnp.dot(q2, p2_ref[...], preferred_element_type=jnp.float32)
    kv2_ref[...] = (jnp.dot(k2, p2k_ref[...], preferred_element_type=jnp.float32)
                    + jnp.dot(v2, p2v_ref[...], preferred_element_type=jnp.float32))
    xr2_ref[...] = jnp.dot(hh, ws2_ref[...], preferred_element_type=jnp.float32) + bs2_ref[...]


def _final_body(a0_ref, a1_ref, xr2_ref, sn_ref, sd_ref, mh_ref, wa_ref, wb_ref,
                out_ref):
    t = a0_ref[...] + a1_ref[...]
    num = jnp.dot(t, sn_ref[...], preferred_element_type=jnp.float32)
    den = jnp.dot(t, sd_ref[...], preferred_element_type=jnp.float32) + 1e-16
    out1 = jnp.dot(num / den, mh_ref[...], preferred_element_type=jnp.float32)
    xr = xr2_ref[...]
    b = jax.nn.sigmoid(out1 * wa_ref[...] + xr * wb_ref[...])
    out_ref[...] = jax.nn.sigmoid(b * xr + (1.0 - b) * out1)


_MESH = plsc.VectorSubcoreMesh(core_axis_name="c", subcore_axis_name="s")


def _make_edge_kernel(cb, qw, kvw, ew, eoff, mw, edge_compute):
    nchunk = NE // cb
    nj_base = nchunk // NWORK
    nj_extra = nchunk - nj_base * NWORK

    @functools.partial(
        pl.kernel,
        out_type=jax.ShapeDtypeStruct((NC * NN, mw), jnp.float32),
        mesh=_MESH,
        scratch_types=[
            pltpu.VMEM((cb,), jnp.int32),
            pltpu.VMEM((cb,), jnp.int32),
            pltpu.VMEM((cb,), jnp.int32),
            pltpu.VMEM((cb,), jnp.int32),
            pltpu.VMEM((cb, qw), jnp.float32),
            pltpu.VMEM((cb, qw), jnp.float32),
            pltpu.VMEM((cb, kvw), jnp.float32),
            pltpu.VMEM((cb, kvw), jnp.float32),
            pltpu.VMEM((cb, ew), jnp.float32),
            pltpu.VMEM((cb, ew), jnp.float32),
            pltpu.VMEM((cb, mw), jnp.float32),
            pltpu.VMEM((cb, mw), jnp.float32),
            pltpu.VMEM_SHARED((NN, mw), jnp.float32),
            pltpu.SemaphoreType.DMA,
            pltpu.SemaphoreType.DMA,
            pltpu.SemaphoreType.DMA,
            pltpu.SemaphoreType.DMA,
        ],
        compiler_params=_SC_PARAMS,
    )
    def _edge(src_hbm, dst_hbm, q_hbm, kv_hbm, e_hbm, z_hbm, out_hbm,
              sv0, sv1, dv0, dv1, qv0, qv1, kvv0, kvv1, ev0, ev1, mv0, mv1,
              acc, gsem0, gsem1, isem0, isem1):
        cid = lax.axis_index("c")
        sid = lax.axis_index("s")
        g = cid * NS + sid
        svs = (sv0, sv1)
        dvs = (dv0, dv1)
        qvs = (qv0, qv1)
        kvvs = (kvv0, kvv1)
        evs = (ev0, ev1)
        mvs = (mv0, mv1)
        gsems = (gsem0, gsem1)
        isems = (isem0, isem1)

        @pl.when(sid < INIT_TILES)
        def _init():
            pltpu.sync_copy(z_hbm.at[pl.ds(sid * RPT, RPT)],
                            acc.at[pl.ds(sid * RPT, RPT)])
        plsc.subcore_barrier()

        nj = jnp.where(g < nj_extra, nj_base + 1, nj_base)

        def ebase(j):
            return pl.multiple_of((g + NWORK * j) * cb, cb)

        def fire_idx(j, b):
            pltpu.async_copy(src_hbm.at[pl.ds(ebase(j), cb)], svs[b], isems[b])
            pltpu.async_copy(dst_hbm.at[pl.ds(ebase(j), cb)], dvs[b], isems[b])

        def wait_idx(j, b):
            pltpu.make_async_copy(src_hbm.at[pl.ds(ebase(j), cb)], svs[b],
                                  isems[b]).wait()
            pltpu.make_async_copy(dst_hbm.at[pl.ds(ebase(j), cb)], dvs[b],
                                  isems[b]).wait()

        def fire_gathers(j, b):
            pltpu.async_copy(q_hbm.at[dvs[b]], qvs[b], gsems[b])
            pltpu.async_copy(kv_hbm.at[svs[b]], kvvs[b], gsems[b])
            pltpu.async_copy(e_hbm.at[pl.ds(ebase(j), cb), pl.ds(eoff, ew)],
                             evs[b], gsems[b])

        def wait_gathers(j, b):
            pltpu.make_async_copy(q_hbm.at[dvs[b]], qvs[b], gsems[b]).wait()
            pltpu.make_async_copy(kv_hbm.at[svs[b]], kvvs[b], gsems[b]).wait()
            pltpu.make_async_copy(e_hbm.at[pl.ds(ebase(j), cb), pl.ds(eoff, ew)],
                                  evs[b], gsems[b]).wait()

        def compute_scatter(b):
            plsc.parallel_loop(0, cb, 1, unroll=4)(
                functools.partial(edge_compute, qvs[b], kvvs[b], evs[b], mvs[b]))
            pltpu.sync_copy(mvs[b], acc.at[dvs[b]], add=True)

        pltpu.sync_copy(src_hbm.at[pl.ds(ebase(0), cb)], sv0)
        pltpu.sync_copy(dst_hbm.at[pl.ds(ebase(0), cb)], dv0)
        fire_gathers(0, 0)
        fire_idx(1, 1)

        def pair_body(p, carry):
            j0 = 2 * p
            j1 = j0 + 1
            wait_gathers(j0, 0)
            wait_idx(j1, 1)
            fire_gathers(j1, 1)
            compute_scatter(0)

            @pl.when(j0 + 2 < nj)
            def _():
                fire_idx(j0 + 2, 0)

            wait_gathers(j1, 1)
            compute_scatter(1)

            @pl.when(j0 + 2 < nj)
            def _():
                wait_idx(j0 + 2, 0)
                fire_gathers(j0 + 2, 0)

            @pl.when(j1 + 2 < nj)
            def _():
                fire_idx(j1 + 2, 1)
            return carry

        lax.fori_loop(0, nj_base // 2, pair_body, 0)

        @pl.when(nj > nj_base)
        def _tail():
            wait_gathers(nj_base, 0)
            compute_scatter(0)

        plsc.subcore_barrier()

        @pl.when(sid < INIT_TILES)
        def _writeback():
            pltpu.sync_copy(acc.at[pl.ds(sid * RPT, RPT)],
                            out_hbm.at[pl.ds(cid * NN + sid * RPT, RPT)])

    return _edge


def _ec1(qv, kvv, ev, mv, ei):
    ecols = []
    alpha = jnp.zeros((NL,), jnp.float32)
    for ci in range(5):
        qc = qv[ei, pl.ds(ci * NL, NL)]
        kc = kvv[ei, pl.ds(ci * NL, NL)]
        ec = ev[ei, pl.ds(ci * NL, NL)]
        ecols.append(ec)
        alpha = alpha + qc * (kc + ec)
    w = jnp.exp(alpha * ISQRT5)
    for ci in range(5):
        vc = kvv[ei, pl.ds(80 + ci * NL, NL)]
        mv[ei, pl.ds(ci * NL, NL)] = (vc + ecols[ci]) * w
    mv[ei, pl.ds(80, NL)] = w


def _ec2(qv, kvv, ev, mv, ei):
    qc = qv[ei, pl.ds(0, NL)]
    kc = kvv[ei, pl.ds(0, NL)]
    vc = kvv[ei, pl.ds(NL, NL)]
    ec = ev[ei, pl.ds(0, NL)]
    w = jnp.exp(qc * (kc + ec))
    mv[ei, pl.ds(0, NL)] = (vc + ec) * w
    mv[ei, pl.ds(NL, NL)] = w


CB1 = 64
CB2 = 128
_edge1 = _make_edge_kernel(CB1, QW, KVW, 80, 0, 96, _ec1)
_edge2 = _make_edge_kernel(CB2, 16, 32, 16, 80, 32, _ec2)


def _perm1():
    p = np.zeros((50, 80), np.float32)
    for h in range(10):
        for c in range(5):
            p[h * 5 + c, c * 16 + h] = 1.0
    return p


def _perm2():
    p = np.zeros((10, 16), np.float32)
    for h in range(10):
        p[h, h] = 1.0
    return p


def _row_spec(width):
    return pl.BlockSpec((NB, width), lambda i: (i, 0))


def _bcast_spec(shape):
    return pl.BlockSpec(shape, lambda i: (0, 0))


def kernel(x, edge_index, edge_attr, batch_idx, Wq1, bq1, Wk1, bk1, Wv1, bv1,
           We1, Ws1, bs1, Wb1, gn_w, gn_b, gn_ms, Wq2, bq2, Wk2, bk2, Wv2, bv2,
           We2, Ws2, bs2, Wb2):
    f32 = jnp.float32
    src = edge_index[0]
    dst = edge_index[1]
    ngrid = NN // NB

    p1 = _perm1()
    p1k = np.concatenate([p1, np.zeros_like(p1)], axis=1)
    p1v = np.concatenate([np.zeros_like(p1), p1], axis=1)
    p2 = _perm2()
    p2k = np.concatenate([p2, np.zeros_like(p2)], axis=1)
    p2v = np.concatenate([np.zeros_like(p2), p2], axis=1)

    p1k = np.zeros((50, KVW), np.float32)
    p1k[:, :80] = p1
    p1v = np.zeros((50, KVW), np.float32)
    p1v[:, 80:160] = p1
    wq_pad = Wq1 @ p1
    bq_pad = (bq1 @ p1)[None, :]
    wkv_pad = Wk1 @ p1k + Wv1 @ p1v
    bkv_pad = (bk1 @ p1k + bv1 @ p1v)[None, :]
    wef = jnp.concatenate([We1 @ p1, We2 @ p2,
                           jnp.zeros((16, EFW - 96), f32)], axis=1)

    q1, kv1, xr1 = pl.pallas_call(
        _node1_body,
        grid=(ngrid,),
        in_specs=[_row_spec(128), _bcast_spec((128, QW)), _bcast_spec((1, QW)),
                  _bcast_spec((128, KVW)), _bcast_spec((1, KVW)),
                  _bcast_spec((128, 5)), _bcast_spec((1, 5))],
        out_specs=(_row_spec(QW), _row_spec(KVW), _row_spec(5)),
        out_shape=(jax.ShapeDtypeStruct((NN, QW), f32),
                   jax.ShapeDtypeStruct((NN, KVW), f32),
                   jax.ShapeDtypeStruct((NN, 5), f32)),
    )(x, wq_pad, bq_pad, wkv_pad, bkv_pad, Ws1, bs1[None, :])

    ef = pl.pallas_call(
        _eproj_body,
        grid=(NE // EB,),
        in_specs=[pl.BlockSpec((16, EB), lambda i: (0, i)),
                  _bcast_spec((16, EFW))],
        out_specs=pl.BlockSpec((EB, EFW), lambda i: (i, 0)),
        out_shape=jax.ShapeDtypeStruct((NE, EFW), f32),
    )(edge_attr.T, wef)

    acc1 = _edge1(src, dst, q1, kv1, ef, jnp.zeros((NN, 96), f32))
    a0 = acc1[:NN]
    a1 = acc1[NN:]

    snn = np.zeros((96, 80), np.float32)
    snn[:80, :] = np.eye(80)
    dbb = np.zeros((96, 80), np.float32)
    for h in range(10):
        for c in range(5):
            dbb[80 + h, c * 16 + h] = 1.0
    mh = np.zeros((80, 5), np.float32)
    for h in range(10):
        for c in range(5):
            mh[c * 16 + h, c] = 0.1
    wba = Wb1[0:5] + Wb1[10:15]
    wbb = Wb1[5:10] - Wb1[10:15]
    h1 = pl.pallas_call(
        _gate1_body,
        grid=(ngrid,),
        in_specs=[_row_spec(96), _row_spec(96), _row_spec(5),
                  _bcast_spec((96, 80)), _bcast_spec((96, 80)),
                  _bcast_spec((80, 5)), _bcast_spec((5, 1)),
                  _bcast_spec((5, 1))],
        out_specs=_row_spec(5),
        out_shape=jax.ShapeDtypeStruct((NN, 5), f32),
    )(a0, a1, xr1, jnp.asarray(snn), jnp.asarray(dbb), jnp.asarray(mh),
      wba, wbb)

    ga, gs = pl.pallas_call(
        _gstats_body,
        out_shape=(jax.ShapeDtypeStruct((64, 5), f32),
                   jax.ShapeDtypeStruct((64, 5), f32)),
    )(h1, batch_idx[None, :], gn_w[None, :], gn_ms[None, :])

    q2, kv2, xr2 = pl.pallas_call(
        _node2_body,
        grid=(ngrid,),
        in_specs=[_row_spec(5), _row_spec(1),
                  _bcast_spec((64, 5)), _bcast_spec((64, 5)),
                  _bcast_spec((1, 5)),
                  _bcast_spec((5, 10)), _bcast_spec((1, 10)),
                  _bcast_spec((5, 10)), _bcast_spec((1, 10)),
                  _bcast_spec((5, 10)), _bcast_spec((1, 10)),
                  _bcast_spec((5, 1)), _bcast_spec((1, 1)),
                  _bcast_spec((10, 16)), _bcast_spec((10, 32)),
                  _bcast_spec((10, 32))],
        out_specs=(_row_spec(16), _row_spec(32), _row_spec(1)),
        out_shape=(jax.ShapeDtypeStruct((NN, 16), f32),
                   jax.ShapeDtypeStruct((NN, 32), f32),
                   jax.ShapeDtypeStruct((NN, 1), f32)),
    )(h1, batch_idx[:, None], ga, gs, gn_b[None, :],
      Wq2, bq2[None, :], Wk2, bk2[None, :], Wv2, bv2[None, :],
      Ws2, bs2[None, :], jnp.asarray(p2), jnp.asarray(p2k), jnp.asarray(p2v))

    acc2 = _edge2(src, dst, q2, kv2, ef, jnp.zeros((NN, 32), f32))
    b0 = acc2[:NN]
    b1 = acc2[NN:]

    sn2 = np.zeros((32, 10), np.float32)
    sn2[:10, :] = np.eye(10)
    sd2 = np.zeros((32, 10), np.float32)
    sd2[16:26, :] = np.eye(10)
    mh2 = np.full((10, 1), 0.1, np.float32)
    wa = (Wb2[0] + Wb2[2])[None, :]
    wb = (Wb2[1] - Wb2[2])[None, :]
    out = pl.pallas_call(
        _final_body,
        grid=(ngrid,),
        in_specs=[_row_spec(32), _row_spec(32), _row_spec(1),
                  _bcast_spec((32, 10)), _bcast_spec((32, 10)),
                  _bcast_spec((10, 1)), _bcast_spec((1, 1)),
                  _bcast_spec((1, 1))],
        out_specs=_row_spec(1),
        out_shape=jax.ShapeDtypeStruct((NN, 1), f32),
    )(b0, b1, xr2, jnp.asarray(sn2), jnp.asarray(sd2), jnp.asarray(mh2),
      wa, wb)
    return out


# speedup vs baseline: 78.0231x; 1.0428x over previous
import functools

import jax
import jax.numpy as jnp
import numpy as np
from jax import lax
from jax.experimental import pallas as pl
from jax.experimental.pallas import tpu as pltpu
from jax.experimental.pallas import tpu_sc as plsc

NN = 10000
NE = 320000
NC, NS, NL = 2, 16, 16
NWORK = NC * NS
CB = 128
NCHUNK = NE // CB
NJ_BASE = NCHUNK // NWORK
NJ_EXTRA = NCHUNK - NJ_BASE * NWORK
INIT_TILES = 10
RPT = NN // INIT_TILES
ISQRT5 = float(1.0 / np.sqrt(5.0))
NB = 2000
EB = 16000
QW = 80
KVW = 160
EFW = 128

_SC_PARAMS = pltpu.CompilerParams(use_tc_tiling_on_sc=False)


def _node1_body(x_ref, wq_ref, bq_ref, wkv_ref, bkv_ref, ws_ref, bs_ref,
                q_ref, kv_ref, xr_ref):
    x = x_ref[...]
    q_ref[...] = jnp.dot(x, wq_ref[...], preferred_element_type=jnp.float32) + bq_ref[...]
    kv_ref[...] = jnp.dot(x, wkv_ref[...], preferred_element_type=jnp.float32) + bkv_ref[...]
    xr_ref[...] = jnp.dot(x, ws_ref[...], preferred_element_type=jnp.float32) + bs_ref[...]


def _eproj_body(eat_ref, w_ref, e_ref):
    e_ref[...] = jax.lax.dot_general(
        eat_ref[...], w_ref[...], (((0,), (0,)), ((), ())),
        preferred_element_type=jnp.float32)


def _gate1_body(a0_ref, a1_ref, xr1_ref, snn_ref, dbb_ref, mh_ref,
                wba_ref, wbb_ref, h1_ref):
    t = a0_ref[...] + a1_ref[...]
    num = jnp.dot(t, snn_ref[...], preferred_element_type=jnp.float32)
    den_b = jnp.dot(t, dbb_ref[...], preferred_element_type=jnp.float32) + 1e-16
    out5 = jnp.dot(num / den_b, mh_ref[...], preferred_element_type=jnp.float32)
    xr = xr1_ref[...]
    b = jax.nn.sigmoid(jnp.dot(out5, wba_ref[...], preferred_element_type=jnp.float32)
                       + jnp.dot(xr, wbb_ref[...], preferred_element_type=jnp.float32))
    h1_ref[...] = b * xr + (1.0 - b) * out5


def _gstats_body(h1_ref, bidxT_ref, gnw_ref, gnms_ref, a_ref, s_ref):
    h1 = h1_ref[...]
    gidT = jax.lax.broadcasted_iota(jnp.int32, (64, NN), 0)
    ohT = (bidxT_ref[...] == gidT).astype(jnp.float32)
    cnt = jnp.sum(ohT, axis=1, keepdims=True)
    inv = 1.0 / jnp.maximum(cnt, 1.0)
    mean_g = jnp.dot(ohT, h1, preferred_element_type=jnp.float32) * inv
    m2_g = jnp.dot(ohT, h1 * h1, preferred_element_type=jnp.float32) * inv
    ms = gnms_ref[...]
    var_g = m2_g - mean_g * mean_g * ms * (2.0 - ms)
    a_ref[...] = mean_g * ms
    s_ref[...] = gnw_ref[...] * jax.lax.rsqrt(var_g + 1e-5)


def _node2_body(h1_ref, bidx_ref, a_ref, s_ref, gnb_ref,
                wq2_ref, bq2_ref, wk2_ref, bk2_ref, wv2_ref, bv2_ref,
                ws2_ref, bs2_ref, p2_ref, p2k_ref, p2v_ref,
                q2_ref, kv2_ref, xr2_ref):
    h1 = h1_ref[...]
    gid = jax.lax.broadcasted_iota(jnp.int32, (NB, 64), 1)
    oh = (bidx_ref[...] == gid).astype(jnp.float32)
    a_n = jnp.dot(oh, a_ref[...], preferred_element_type=jnp.float32)
    s_n = jnp.dot(oh, s_ref[...], preferred_element_type=jnp.float32)
    hh = (h1 - a_n) * s_n + gnb_ref[...]
    hh = jnp.maximum(hh, 0.0)
    q2 = jnp.dot(hh, wq2_ref[...], preferred_element_type=jnp.float32) + bq2_ref[...]
    k2 = jnp.dot(hh, wk2_ref[...], preferred_element_type=jnp.float32) + bk2_ref[...]
    v2 = jnp.dot(hh, wv2_ref[...], preferred_element_type=jnp.float32) + bv2_ref[...]
    q2_ref[...] = jnp.dot(q2, p2_ref[...], preferred_element_type=jnp.float32)
    kv2_ref[...] = (jnp.dot(k2, p2k_ref[...], preferred_element_type=jnp.float32)
                    + jnp.dot(v2, p2v_ref[...], preferred_element_type=jnp.float32))
    xr2_ref[...] = jnp.dot(hh, ws2_ref[...], preferred_element_type=jnp.float32) + bs2_ref[...]


def _final_body(a0_ref, a1_ref, xr2_ref, sn_ref, sd_ref, mh_ref, wa_ref, wb_ref,
                out_ref):
    t = a0_ref[...] + a1_ref[...]
    num = jnp.dot(t, sn_ref[...], preferred_element_type=jnp.float32)
    den = jnp.dot(t, sd_ref[...], preferred_element_type=jnp.float32) + 1e-16
    out1 = jnp.dot(num / den, mh_ref[...], preferred_element_type=jnp.float32)
    xr = xr2_ref[...]
    b = jax.nn.sigmoid(out1 * wa_ref[...] + xr * wb_ref[...])
    out_ref[...] = jax.nn.sigmoid(b * xr + (1.0 - b) * out1)


_MESH = plsc.VectorSubcoreMesh(core_axis_name="c", subcore_axis_name="s")


def _make_edge_kernel(cb, qw, kvw, ew, eoff, mw, edge_compute):
    nchunk = NE // cb
    nj_base = nchunk // NWORK
    nj_extra = nchunk - nj_base * NWORK

    @functools.partial(
        pl.kernel,
        out_type=jax.ShapeDtypeStruct((NC * NN, mw), jnp.float32),
        mesh=_MESH,
        scratch_types=[
            pltpu.VMEM((cb,), jnp.int32),
            pltpu.VMEM((cb,), jnp.int32),
            pltpu.VMEM((cb,), jnp.int32),
            pltpu.VMEM((cb,), jnp.int32),
            pltpu.VMEM((cb,), jnp.int32),
            pltpu.VMEM((cb,), jnp.int32),
            pltpu.VMEM((cb, qw), jnp.float32),
            pltpu.VMEM((cb, qw), jnp.float32),
            pltpu.VMEM((cb, kvw), jnp.float32),
            pltpu.VMEM((cb, kvw), jnp.float32),
            pltpu.VMEM((cb, ew), jnp.float32),
            pltpu.VMEM((cb, ew), jnp.float32),
            pltpu.VMEM((cb, mw), jnp.float32),
            pltpu.VMEM((cb, mw), jnp.float32),
            pltpu.VMEM_SHARED((NN, mw), jnp.float32),
            pltpu.SemaphoreType.DMA,
            pltpu.SemaphoreType.DMA,
            pltpu.SemaphoreType.DMA,
            pltpu.SemaphoreType.DMA,
            pltpu.SemaphoreType.DMA,
            pltpu.SemaphoreType.DMA,
        ],
        compiler_params=_SC_PARAMS,
    )
    def _edge(src_hbm, dst_hbm, q_hbm, kv_hbm, e_hbm, z_hbm, out_hbm,
              sv0, sv1, dv0, dv1, ds0, ds1, qv0, qv1, kvv0, kvv1, ev0, ev1,
              mv0, mv1, acc, gsem0, gsem1, isem0, isem1, ssem0, ssem1):
        cid = lax.axis_index("c")
        sid = lax.axis_index("s")
        g = cid * NS + sid
        svs = (sv0, sv1)
        dvs = (dv0, dv1)
        dscats = (ds0, ds1)
        qvs = (qv0, qv1)
        kvvs = (kvv0, kvv1)
        evs = (ev0, ev1)
        mvs = (mv0, mv1)
        gsems = (gsem0, gsem1)
        isems = (isem0, isem1)
        ssems = (ssem0, ssem1)

        @pl.when(sid < INIT_TILES)
        def _init():
            pltpu.sync_copy(z_hbm.at[pl.ds(sid * RPT, RPT)],
                            acc.at[pl.ds(sid * RPT, RPT)])
        plsc.subcore_barrier()

        nj = jnp.where(g < nj_extra, nj_base + 1, nj_base)

        def ebase(j):
            return pl.multiple_of((g + NWORK * j) * cb, cb)

        def fire_idx(j, b):
            pltpu.async_copy(src_hbm.at[pl.ds(ebase(j), cb)], svs[b], isems[b])
            pltpu.async_copy(dst_hbm.at[pl.ds(ebase(j), cb)], dvs[b], isems[b])

        def wait_idx(j, b):
            pltpu.make_async_copy(src_hbm.at[pl.ds(ebase(j), cb)], svs[b],
                                  isems[b]).wait()
            pltpu.make_async_copy(dst_hbm.at[pl.ds(ebase(j), cb)], dvs[b],
                                  isems[b]).wait()

        def fire_gathers(j, b):
            pltpu.async_copy(q_hbm.at[dvs[b]], qvs[b], gsems[b])
            pltpu.async_copy(kv_hbm.at[svs[b]], kvvs[b], gsems[b])
            pltpu.async_copy(e_hbm.at[pl.ds(ebase(j), cb), pl.ds(eoff, ew)],
                             evs[b], gsems[b])

        def wait_gathers(j, b):
            pltpu.make_async_copy(q_hbm.at[dvs[b]], qvs[b], gsems[b]).wait()
            pltpu.make_async_copy(kv_hbm.at[svs[b]], kvvs[b], gsems[b]).wait()
            pltpu.make_async_copy(e_hbm.at[pl.ds(ebase(j), cb), pl.ds(eoff, ew)],
                                  evs[b], gsems[b]).wait()

        def compute_scatter(b, pending):
            @pl.when(pending)
            def _():
                pltpu.make_async_copy(mvs[b], acc.at[dscats[b]], ssems[b]).wait()
            plsc.parallel_loop(0, cb, 1, unroll=8)(
                functools.partial(edge_compute, qvs[b], kvvs[b], evs[b], mvs[b]))
            for o in range(cb // NL):
                dscats[b][pl.ds(o * NL, NL)] = dvs[b][pl.ds(o * NL, NL)]
            pltpu.async_copy(mvs[b], acc.at[dscats[b]], ssems[b], add=True)

        pltpu.sync_copy(src_hbm.at[pl.ds(ebase(0), cb)], sv0)
        pltpu.sync_copy(dst_hbm.at[pl.ds(ebase(0), cb)], dv0)
        fire_gathers(0, 0)
        fire_idx(1, 1)

        def pair_body(p, carry):
            j0 = 2 * p
            j1 = j0 + 1
            wait_gathers(j0, 0)
            wait_idx(j1, 1)
            fire_gathers(j1, 1)
            compute_scatter(0, j0 >= 2)

            @pl.when(j0 + 2 < nj)
            def _():
                fire_idx(j0 + 2, 0)

            wait_gathers(j1, 1)
            compute_scatter(1, j1 >= 3)

            @pl.when(j0 + 2 < nj)
            def _():
                wait_idx(j0 + 2, 0)
                fire_gathers(j0 + 2, 0)

            @pl.when(j1 + 2 < nj)
            def _():
                fire_idx(j1 + 2, 1)
            return carry

        lax.fori_loop(0, nj_base // 2, pair_body, 0)

        @pl.when(nj > nj_base)
        def _tail():
            wait_gathers(nj_base, 0)
            compute_scatter(0, nj_base >= 2)

        pltpu.make_async_copy(mvs[0], acc.at[dscats[0]], ssems[0]).wait()
        pltpu.make_async_copy(mvs[1], acc.at[dscats[1]], ssems[1]).wait()
        plsc.subcore_barrier()

        @pl.when(sid < INIT_TILES)
        def _writeback():
            pltpu.sync_copy(acc.at[pl.ds(sid * RPT, RPT)],
                            out_hbm.at[pl.ds(cid * NN + sid * RPT, RPT)])

    return _edge


def _ec1(qv, kvv, ev, mv, ei):
    ecols = []
    alpha = jnp.zeros((NL,), jnp.float32)
    for ci in range(5):
        qc = qv[ei, pl.ds(ci * NL, NL)]
        kc = kvv[ei, pl.ds(ci * NL, NL)]
        ec = ev[ei, pl.ds(ci * NL, NL)]
        ecols.append(ec)
        alpha = alpha + qc * (kc + ec)
    w = jnp.exp(alpha * ISQRT5)
    for ci in range(5):
        vc = kvv[ei, pl.ds(80 + ci * NL, NL)]
        mv[ei, pl.ds(ci * NL, NL)] = (vc + ecols[ci]) * w
    mv[ei, pl.ds(80, NL)] = w


def _ec2(qv, kvv, ev, mv, ei):
    qc = qv[ei, pl.ds(0, NL)]
    kc = kvv[ei, pl.ds(0, NL)]
    vc = kvv[ei, pl.ds(NL, NL)]
    ec = ev[ei, pl.ds(0, NL)]
    w = jnp.exp(qc * (kc + ec))
    mv[ei, pl.ds(0, NL)] = (vc + ec) * w
    mv[ei, pl.ds(NL, NL)] = w


CB1 = 64
CB2 = 128
_edge1 = _make_edge_kernel(CB1, QW, KVW, 80, 0, 96, _ec1)
_edge2 = _make_edge_kernel(CB2, 16, 32, 16, 80, 32, _ec2)


def _perm1():
    p = np.zeros((50, 80), np.float32)
    for h in range(10):
        for c in range(5):
            p[h * 5 + c, c * 16 + h] = 1.0
    return p


def _perm2():
    p = np.zeros((10, 16), np.float32)
    for h in range(10):
        p[h, h] = 1.0
    return p


def _row_spec(width):
    return pl.BlockSpec((NB, width), lambda i: (i, 0))


def _bcast_spec(shape):
    return pl.BlockSpec(shape, lambda i: (0, 0))


def kernel(x, edge_index, edge_attr, batch_idx, Wq1, bq1, Wk1, bk1, Wv1, bv1,
           We1, Ws1, bs1, Wb1, gn_w, gn_b, gn_ms, Wq2, bq2, Wk2, bk2, Wv2, bv2,
           We2, Ws2, bs2, Wb2):
    f32 = jnp.float32
    src = edge_index[0]
    dst = edge_index[1]
    ngrid = NN // NB

    p1 = _perm1()
    p1k = np.concatenate([p1, np.zeros_like(p1)], axis=1)
    p1v = np.concatenate([np.zeros_like(p1), p1], axis=1)
    p2 = _perm2()
    p2k = np.concatenate([p2, np.zeros_like(p2)], axis=1)
    p2v = np.concatenate([np.zeros_like(p2), p2], axis=1)

    p1k = np.zeros((50, KVW), np.float32)
    p1k[:, :80] = p1
    p1v = np.zeros((50, KVW), np.float32)
    p1v[:, 80:160] = p1
    wq_pad = Wq1 @ p1
    bq_pad = (bq1 @ p1)[None, :]
    wkv_pad = Wk1 @ p1k + Wv1 @ p1v
    bkv_pad = (bk1 @ p1k + bv1 @ p1v)[None, :]
    wef = jnp.concatenate([We1 @ p1, We2 @ p2,
                           jnp.zeros((16, EFW - 96), f32)], axis=1)

    q1, kv1, xr1 = pl.pallas_call(
        _node1_body,
        grid=(ngrid,),
        in_specs=[_row_spec(128), _bcast_spec((128, QW)), _bcast_spec((1, QW)),
                  _bcast_spec((128, KVW)), _bcast_spec((1, KVW)),
                  _bcast_spec((128, 5)), _bcast_spec((1, 5))],
        out_specs=(_row_spec(QW), _row_spec(KVW), _row_spec(5)),
        out_shape=(jax.ShapeDtypeStruct((NN, QW), f32),
                   jax.ShapeDtypeStruct((NN, KVW), f32),
                   jax.ShapeDtypeStruct((NN, 5), f32)),
    )(x, wq_pad, bq_pad, wkv_pad, bkv_pad, Ws1, bs1[None, :])

    ef = pl.pallas_call(
        _eproj_body,
        grid=(NE // EB,),
        in_specs=[pl.BlockSpec((16, EB), lambda i: (0, i)),
                  _bcast_spec((16, EFW))],
        out_specs=pl.BlockSpec((EB, EFW), lambda i: (i, 0)),
        out_shape=jax.ShapeDtypeStruct((NE, EFW), f32),
    )(edge_attr.T, wef)

    acc1 = _edge1(src, dst, q1, kv1, ef, jnp.zeros((NN, 96), f32))
    a0 = acc1[:NN]
    a1 = acc1[NN:]

    snn = np.zeros((96, 80), np.float32)
    snn[:80, :] = np.eye(80)
    dbb = np.zeros((96, 80), np.float32)
    for h in range(10):
        for c in range(5):
            dbb[80 + h, c * 16 + h] = 1.0
    mh = np.zeros((80, 5), np.float32)
    for h in range(10):
        for c in range(5):
            mh[c * 16 + h, c] = 0.1
    wba = Wb1[0:5] + Wb1[10:15]
    wbb = Wb1[5:10] - Wb1[10:15]
    h1 = pl.pallas_call(
        _gate1_body,
        grid=(ngrid,),
        in_specs=[_row_spec(96), _row_spec(96), _row_spec(5),
                  _bcast_spec((96, 80)), _bcast_spec((96, 80)),
                  _bcast_spec((80, 5)), _bcast_spec((5, 1)),
                  _bcast_spec((5, 1))],
        out_specs=_row_spec(5),
        out_shape=jax.ShapeDtypeStruct((NN, 5), f32),
    )(a0, a1, xr1, jnp.asarray(snn), jnp.asarray(dbb), jnp.asarray(mh),
      wba, wbb)

    ga, gs = pl.pallas_call(
        _gstats_body,
        out_shape=(jax.ShapeDtypeStruct((64, 5), f32),
                   jax.ShapeDtypeStruct((64, 5), f32)),
    )(h1, batch_idx[None, :], gn_w[None, :], gn_ms[None, :])

    q2, kv2, xr2 = pl.pallas_call(
        _node2_body,
        grid=(ngrid,),
        in_specs=[_row_spec(5), _row_spec(1),
                  _bcast_spec((64, 5)), _bcast_spec((64, 5)),
                  _bcast_spec((1, 5)),
                  _bcast_spec((5, 10)), _bcast_spec((1, 10)),
                  _bcast_spec((5, 10)), _bcast_spec((1, 10)),
                  _bcast_spec((5, 10)), _bcast_spec((1, 10)),
                  _bcast_spec((5, 1)), _bcast_spec((1, 1)),
                  _bcast_spec((10, 16)), _bcast_spec((10, 32)),
                  _bcast_spec((10, 32))],
        out_specs=(_row_spec(16), _row_spec(32), _row_spec(1)),
        out_shape=(jax.ShapeDtypeStruct((NN, 16), f32),
                   jax.ShapeDtypeStruct((NN, 32), f32),
                   jax.ShapeDtypeStruct((NN, 1), f32)),
    )(h1, batch_idx[:, None], ga, gs, gn_b[None, :],
      Wq2, bq2[None, :], Wk2, bk2[None, :], Wv2, bv2[None, :],
      Ws2, bs2[None, :], jnp.asarray(p2), jnp.asarray(p2k), jnp.asarray(p2v))

    acc2 = _edge2(src, dst, q2, kv2, ef, jnp.zeros((NN, 32), f32))
    b0 = acc2[:NN]
    b1 = acc2[NN:]

    sn2 = np.zeros((32, 10), np.float32)
    sn2[:10, :] = np.eye(10)
    sd2 = np.zeros((32, 10), np.float32)
    sd2[16:26, :] = np.eye(10)
    mh2 = np.full((10, 1), 0.1, np.float32)
    wa = (Wb2[0] + Wb2[2])[None, :]
    wb = (Wb2[1] - Wb2[2])[None, :]
    out = pl.pallas_call(
        _final_body,
        grid=(ngrid,),
        in_specs=[_row_spec(32), _row_spec(32), _row_spec(1),
                  _bcast_spec((32, 10)), _bcast_spec((32, 10)),
                  _bcast_spec((10, 1)), _bcast_spec((1, 1)),
                  _bcast_spec((1, 1))],
        out_specs=_row_spec(1),
        out_shape=jax.ShapeDtypeStruct((NN, 1), f32),
    )(b0, b1, xr2, jnp.asarray(sn2), jnp.asarray(sd2), jnp.asarray(mh2),
      wa, wb)
    return out


# speedup vs baseline: 79.8117x; 1.0229x over previous
import functools

import jax
import jax.numpy as jnp
import numpy as np
from jax import lax
from jax.experimental import pallas as pl
from jax.experimental.pallas import tpu as pltpu
from jax.experimental.pallas import tpu_sc as plsc

NN = 10000
NE = 320000
NC, NS, NL = 2, 16, 16
NWORK = NC * NS
CB = 128
NCHUNK = NE // CB
NJ_BASE = NCHUNK // NWORK
NJ_EXTRA = NCHUNK - NJ_BASE * NWORK
INIT_TILES = 10
RPT = NN // INIT_TILES
ISQRT5 = float(1.0 / np.sqrt(5.0))
NB = 2000
EB = 16000
QW = 80
KVW = 160
EFW = 128

_SC_PARAMS = pltpu.CompilerParams(use_tc_tiling_on_sc=False)


def _node1_body(x_ref, wq_ref, bq_ref, wkv_ref, bkv_ref, ws_ref, bs_ref,
                q_ref, kv_ref, xr_ref):
    x = x_ref[...]
    q_ref[...] = jnp.dot(x, wq_ref[...], preferred_element_type=jnp.float32) + bq_ref[...]
    kv_ref[...] = jnp.dot(x, wkv_ref[...], preferred_element_type=jnp.float32) + bkv_ref[...]
    xr_ref[...] = jnp.dot(x, ws_ref[...], preferred_element_type=jnp.float32) + bs_ref[...]


def _eproj_body(eat_ref, w_ref, e_ref):
    e_ref[...] = jax.lax.dot_general(
        eat_ref[...], w_ref[...], (((0,), (0,)), ((), ())),
        preferred_element_type=jnp.float32)


def _gate1_body(a0_ref, a1_ref, xr1_ref, snn_ref, dbb_ref, mh_ref,
                wba_ref, wbb_ref, h1_ref):
    t = a0_ref[...] + a1_ref[...]
    num = jnp.dot(t, snn_ref[...], preferred_element_type=jnp.float32)
    den_b = jnp.dot(t, dbb_ref[...], preferred_element_type=jnp.float32) + 1e-16
    out5 = jnp.dot(num / den_b, mh_ref[...], preferred_element_type=jnp.float32)
    xr = xr1_ref[...]
    b = jax.nn.sigmoid(jnp.dot(out5, wba_ref[...], preferred_element_type=jnp.float32)
                       + jnp.dot(xr, wbb_ref[...], preferred_element_type=jnp.float32))
    h1_ref[...] = b * xr + (1.0 - b) * out5


def _gstats_body(h1_ref, bidxT_ref, gnw_ref, gnms_ref, a_ref, s_ref):
    h1 = h1_ref[...]
    gidT = jax.lax.broadcasted_iota(jnp.int32, (64, NN), 0)
    ohT = (bidxT_ref[...] == gidT).astype(jnp.float32)
    cnt = jnp.sum(ohT, axis=1, keepdims=True)
    inv = 1.0 / jnp.maximum(cnt, 1.0)
    mean_g = jnp.dot(ohT, h1, preferred_element_type=jnp.float32) * inv
    m2_g = jnp.dot(ohT, h1 * h1, preferred_element_type=jnp.float32) * inv
    ms = gnms_ref[...]
    var_g = m2_g - mean_g * mean_g * ms * (2.0 - ms)
    a_ref[...] = mean_g * ms
    s_ref[...] = gnw_ref[...] * jax.lax.rsqrt(var_g + 1e-5)


def _node2_body(h1_ref, bidx_ref, a_ref, s_ref, gnb_ref,
                wq2_ref, bq2_ref, wk2_ref, bk2_ref, wv2_ref, bv2_ref,
                ws2_ref, bs2_ref, p2_ref, p2k_ref, p2v_ref,
                q2_ref, kv2_ref, xr2_ref):
    h1 = h1_ref[...]
    gid = jax.lax.broadcasted_iota(jnp.int32, (NB, 64), 1)
    oh = (bidx_ref[...] == gid).astype(jnp.float32)
    a_n = jnp.dot(oh, a_ref[...], preferred_element_type=jnp.float32)
    s_n = jnp.dot(oh, s_ref[...], preferred_element_type=jnp.float32)
    hh = (h1 - a_n) * s_n + gnb_ref[...]
    hh = jnp.maximum(hh, 0.0)
    q2 = jnp.dot(hh, wq2_ref[...], preferred_element_type=jnp.float32) + bq2_ref[...]
    k2 = jnp.dot(hh, wk2_ref[...], preferred_element_type=jnp.float32) + bk2_ref[...]
    v2 = jnp.dot(hh, wv2_ref[...], preferred_element_type=jnp.float32) + bv2_ref[...]
    q2_ref[...] = jnp.dot(q2, p2_ref[...], preferred_element_type=jnp.float32)
    kv2_ref[...] = (jnp.dot(k2, p2k_ref[...], preferred_element_type=jnp.float32)
                    + jnp.dot(v2, p2v_ref[...], preferred_element_type=jnp.float32))
    xr2_ref[...] = jnp.dot(hh, ws2_ref[...], preferred_element_type=jnp.float32) + bs2_ref[...]


def _final_body(a0_ref, a1_ref, xr2_ref, sn_ref, sd_ref, mh_ref, wa_ref, wb_ref,
                out_ref):
    t = a0_ref[...] + a1_ref[...]
    num = jnp.dot(t, sn_ref[...], preferred_element_type=jnp.float32)
    den = jnp.dot(t, sd_ref[...], preferred_element_type=jnp.float32) + 1e-16
    out1 = jnp.dot(num / den, mh_ref[...], preferred_element_type=jnp.float32)
    xr = xr2_ref[...]
    b = jax.nn.sigmoid(out1 * wa_ref[...] + xr * wb_ref[...])
    out_ref[...] = jax.nn.sigmoid(b * xr + (1.0 - b) * out1)


_MESH = plsc.VectorSubcoreMesh(core_axis_name="c", subcore_axis_name="s")


def _make_edge_kernel(cb, qw, kvw, ew, eoff, mw, edge_compute):
    nchunk = NE // cb
    nj_base = nchunk // NWORK
    nj_extra = nchunk - nj_base * NWORK

    @functools.partial(
        pl.kernel,
        out_type=jax.ShapeDtypeStruct((NC * NN, mw), jnp.float32),
        mesh=_MESH,
        scratch_types=[
            pltpu.VMEM((cb,), jnp.int32),
            pltpu.VMEM((cb,), jnp.int32),
            pltpu.VMEM((cb,), jnp.int32),
            pltpu.VMEM((cb,), jnp.int32),
            pltpu.VMEM((cb,), jnp.int32),
            pltpu.VMEM((cb,), jnp.int32),
            pltpu.VMEM((cb, qw), jnp.float32),
            pltpu.VMEM((cb, qw), jnp.float32),
            pltpu.VMEM((cb, kvw), jnp.float32),
            pltpu.VMEM((cb, kvw), jnp.float32),
            pltpu.VMEM((cb, ew), jnp.float32),
            pltpu.VMEM((cb, ew), jnp.float32),
            pltpu.VMEM((cb, mw), jnp.float32),
            pltpu.VMEM((cb, mw), jnp.float32),
            pltpu.VMEM_SHARED((NN, mw), jnp.float32),
            pltpu.SemaphoreType.DMA,
            pltpu.SemaphoreType.DMA,
            pltpu.SemaphoreType.DMA,
            pltpu.SemaphoreType.DMA,
            pltpu.SemaphoreType.DMA,
            pltpu.SemaphoreType.DMA,
        ],
        compiler_params=_SC_PARAMS,
    )
    def _edge(src_hbm, dst_hbm, q_hbm, kv_hbm, e_hbm, z_hbm, out_hbm,
              sv0, sv1, dv0, dv1, ds0, ds1, qv0, qv1, kvv0, kvv1, ev0, ev1,
              mv0, mv1, acc, gsem0, gsem1, isem0, isem1, ssem0, ssem1):
        cid = lax.axis_index("c")
        sid = lax.axis_index("s")
        g = cid * NS + sid
        svs = (sv0, sv1)
        dvs = (dv0, dv1)
        dscats = (ds0, ds1)
        qvs = (qv0, qv1)
        kvvs = (kvv0, kvv1)
        evs = (ev0, ev1)
        mvs = (mv0, mv1)
        gsems = (gsem0, gsem1)
        isems = (isem0, isem1)
        ssems = (ssem0, ssem1)

        @pl.when(sid < INIT_TILES)
        def _init():
            pltpu.sync_copy(z_hbm.at[pl.ds(sid * RPT, RPT)],
                            acc.at[pl.ds(sid * RPT, RPT)])
        plsc.subcore_barrier()

        nj = jnp.where(g < nj_extra, nj_base + 1, nj_base)

        def ebase(j):
            return pl.multiple_of((g + NWORK * j) * cb, cb)

        def fire_idx(j, b):
            pltpu.async_copy(src_hbm.at[pl.ds(ebase(j), cb)], svs[b], isems[b])
            pltpu.async_copy(dst_hbm.at[pl.ds(ebase(j), cb)], dvs[b], isems[b])

        def wait_idx(j, b):
            pltpu.make_async_copy(src_hbm.at[pl.ds(ebase(j), cb)], svs[b],
                                  isems[b]).wait()
            pltpu.make_async_copy(dst_hbm.at[pl.ds(ebase(j), cb)], dvs[b],
                                  isems[b]).wait()

        def fire_gathers(j, b):
            pltpu.async_copy(q_hbm.at[dvs[b]], qvs[b], gsems[b])
            pltpu.async_copy(kv_hbm.at[svs[b]], kvvs[b], gsems[b])
            pltpu.async_copy(e_hbm.at[pl.ds(ebase(j), cb), pl.ds(eoff, ew)],
                             evs[b], gsems[b])

        def wait_gathers(j, b):
            pltpu.make_async_copy(q_hbm.at[dvs[b]], qvs[b], gsems[b]).wait()
            pltpu.make_async_copy(kv_hbm.at[svs[b]], kvvs[b], gsems[b]).wait()
            pltpu.make_async_copy(e_hbm.at[pl.ds(ebase(j), cb), pl.ds(eoff, ew)],
                                  evs[b], gsems[b]).wait()

        def compute_scatter(b, pending):
            @pl.when(pending)
            def _():
                pltpu.make_async_copy(mvs[b], acc.at[dscats[b]], ssems[b]).wait()
            plsc.parallel_loop(0, cb, 1, unroll=8)(
                functools.partial(edge_compute, qvs[b], kvvs[b], evs[b], mvs[b]))
            for o in range(cb // NL):
                dscats[b][pl.ds(o * NL, NL)] = dvs[b][pl.ds(o * NL, NL)]
            pltpu.async_copy(mvs[b], acc.at[dscats[b]], ssems[b], add=True)

        pltpu.sync_copy(src_hbm.at[pl.ds(ebase(0), cb)], sv0)
        pltpu.sync_copy(dst_hbm.at[pl.ds(ebase(0), cb)], dv0)
        fire_gathers(0, 0)
        fire_idx(1, 1)

        def pair_body(p, carry):
            j0 = 2 * p
            j1 = j0 + 1
            wait_gathers(j0, 0)
            wait_idx(j1, 1)
            fire_gathers(j1, 1)
            compute_scatter(0, j0 >= 2)

            @pl.when(j0 + 2 < nj)
            def _():
                fire_idx(j0 + 2, 0)

            wait_gathers(j1, 1)
            compute_scatter(1, j1 >= 3)

            @pl.when(j0 + 2 < nj)
            def _():
                wait_idx(j0 + 2, 0)
                fire_gathers(j0 + 2, 0)

            @pl.when(j1 + 2 < nj)
            def _():
                fire_idx(j1 + 2, 1)
            return carry

        lax.fori_loop(0, nj_base // 2, pair_body, 0)

        @pl.when(nj > nj_base)
        def _tail():
            wait_gathers(nj_base, 0)
            compute_scatter(0, nj_base >= 2)

        pltpu.make_async_copy(mvs[0], acc.at[dscats[0]], ssems[0]).wait()
        pltpu.make_async_copy(mvs[1], acc.at[dscats[1]], ssems[1]).wait()
        plsc.subcore_barrier()

        @pl.when(sid < INIT_TILES)
        def _writeback():
            pltpu.sync_copy(acc.at[pl.ds(sid * RPT, RPT)],
                            out_hbm.at[pl.ds(cid * NN + sid * RPT, RPT)])

    return _edge


def _ec1(qv, kvv, ev, mv, ei):
    ecols = []
    alpha = jnp.zeros((NL,), jnp.float32)
    for ci in range(5):
        qc = qv[ei, pl.ds(ci * NL, NL)]
        kc = kvv[ei, pl.ds(ci * NL, NL)]
        ec = ev[ei, pl.ds(ci * NL, NL)]
        ecols.append(ec)
        alpha = alpha + qc * (kc + ec)
    w = jnp.exp(alpha * ISQRT5)
    for ci in range(5):
        vc = kvv[ei, pl.ds(80 + ci * NL, NL)]
        mv[ei, pl.ds(ci * NL, NL)] = (vc + ecols[ci]) * w
    mv[ei, pl.ds(80, NL)] = w


def _ec2(qv, kvv, ev, mv, ei):
    qc = qv[ei, pl.ds(0, NL)]
    kc = kvv[ei, pl.ds(0, NL)]
    vc = kvv[ei, pl.ds(NL, NL)]
    ec = ev[ei, pl.ds(0, NL)]
    w = jnp.exp(qc * (kc + ec))
    mv[ei, pl.ds(0, NL)] = (vc + ec) * w
    mv[ei, pl.ds(NL, NL)] = w


CB1 = 64
CB2 = 128
_edge1 = _make_edge_kernel(CB1, QW, KVW, 80, 0, 96, _ec1)
_edge2 = _make_edge_kernel(CB2, 16, 32, 16, 80, 32, _ec2)


def _perm1():
    p = np.zeros((50, 80), np.float32)
    for h in range(10):
        for c in range(5):
            p[h * 5 + c, c * 16 + h] = 1.0
    return p


def _perm2():
    p = np.zeros((10, 16), np.float32)
    for h in range(10):
        p[h, h] = 1.0
    return p


def _row_spec(width):
    return pl.BlockSpec((NB, width), lambda i: (i, 0))


def _bcast_spec(shape):
    return pl.BlockSpec(shape, lambda i: (0, 0))


def kernel(x, edge_index, edge_attr, batch_idx, Wq1, bq1, Wk1, bk1, Wv1, bv1,
           We1, Ws1, bs1, Wb1, gn_w, gn_b, gn_ms, Wq2, bq2, Wk2, bk2, Wv2, bv2,
           We2, Ws2, bs2, Wb2):
    f32 = jnp.float32
    src = edge_index[0]
    dst = edge_index[1]
    ngrid = NN // NB

    p1 = _perm1()
    p1k = np.concatenate([p1, np.zeros_like(p1)], axis=1)
    p1v = np.concatenate([np.zeros_like(p1), p1], axis=1)
    p2 = _perm2()
    p2k = np.concatenate([p2, np.zeros_like(p2)], axis=1)
    p2v = np.concatenate([np.zeros_like(p2), p2], axis=1)

    p1k = np.zeros((50, KVW), np.float32)
    p1k[:, :80] = p1
    p1v = np.zeros((50, KVW), np.float32)
    p1v[:, 80:160] = p1
    wq_pad = Wq1 @ p1
    bq_pad = (bq1 @ p1)[None, :]
    wkv_pad = Wk1 @ p1k + Wv1 @ p1v
    bkv_pad = (bk1 @ p1k + bv1 @ p1v)[None, :]
    wef = jnp.concatenate([We1 @ p1, We2 @ p2,
                           jnp.zeros((16, EFW - 96), f32)], axis=1)

    q1, kv1, xr1 = pl.pallas_call(
        _node1_body,
        grid=(ngrid,),
        in_specs=[_row_spec(128), _bcast_spec((128, QW)), _bcast_spec((1, QW)),
                  _bcast_spec((128, KVW)), _bcast_spec((1, KVW)),
                  _bcast_spec((128, 5)), _bcast_spec((1, 5))],
        out_specs=(_row_spec(QW), _row_spec(KVW), _row_spec(5)),
        out_shape=(jax.ShapeDtypeStruct((NN, QW), f32),
                   jax.ShapeDtypeStruct((NN, KVW), f32),
                   jax.ShapeDtypeStruct((NN, 5), f32)),
    )(x, wq_pad, bq_pad, wkv_pad, bkv_pad, Ws1, bs1[None, :])

    ef = pl.pallas_call(
        _eproj_body,
        grid=(NE // EB,),
        in_specs=[pl.BlockSpec((16, EB), lambda i: (0, i)),
                  _bcast_spec((16, EFW))],
        out_specs=pl.BlockSpec((EB, EFW), lambda i: (i, 0)),
        out_shape=jax.ShapeDtypeStruct((NE, EFW), f32),
    )(edge_attr.T, wef)

    acc1 = _edge1(src, dst, q1, kv1, ef, jnp.zeros((NN, 96), f32))

    snn = np.zeros((96, 80), np.float32)
    snn[:80, :] = np.eye(80)
    dbb = np.zeros((96, 80), np.float32)
    for h in range(10):
        for c in range(5):
            dbb[80 + h, c * 16 + h] = 1.0
    mh = np.zeros((80, 5), np.float32)
    for h in range(10):
        for c in range(5):
            mh[c * 16 + h, c] = 0.1
    wba = Wb1[0:5] + Wb1[10:15]
    wbb = Wb1[5:10] - Wb1[10:15]
    nblk = NN // NB
    h1 = pl.pallas_call(
        _gate1_body,
        grid=(ngrid,),
        in_specs=[pl.BlockSpec((NB, 96), lambda i: (i, 0)),
                  pl.BlockSpec((NB, 96), lambda i, n=nblk: (i + n, 0)),
                  _row_spec(5),
                  _bcast_spec((96, 80)), _bcast_spec((96, 80)),
                  _bcast_spec((80, 5)), _bcast_spec((5, 1)),
                  _bcast_spec((5, 1))],
        out_specs=_row_spec(5),
        out_shape=jax.ShapeDtypeStruct((NN, 5), f32),
    )(acc1, acc1, xr1, jnp.asarray(snn), jnp.asarray(dbb), jnp.asarray(mh),
      wba, wbb)

    ga, gs = pl.pallas_call(
        _gstats_body,
        out_shape=(jax.ShapeDtypeStruct((64, 5), f32),
                   jax.ShapeDtypeStruct((64, 5), f32)),
    )(h1, batch_idx[None, :], gn_w[None, :], gn_ms[None, :])

    q2, kv2, xr2 = pl.pallas_call(
        _node2_body,
        grid=(ngrid,),
        in_specs=[_row_spec(5), _row_spec(1),
                  _bcast_spec((64, 5)), _bcast_spec((64, 5)),
                  _bcast_spec((1, 5)),
                  _bcast_spec((5, 10)), _bcast_spec((1, 10)),
                  _bcast_spec((5, 10)), _bcast_spec((1, 10)),
                  _bcast_spec((5, 10)), _bcast_spec((1, 10)),
                  _bcast_spec((5, 1)), _bcast_spec((1, 1)),
                  _bcast_spec((10, 16)), _bcast_spec((10, 32)),
                  _bcast_spec((10, 32))],
        out_specs=(_row_spec(16), _row_spec(32), _row_spec(1)),
        out_shape=(jax.ShapeDtypeStruct((NN, 16), f32),
                   jax.ShapeDtypeStruct((NN, 32), f32),
                   jax.ShapeDtypeStruct((NN, 1), f32)),
    )(h1, batch_idx[:, None], ga, gs, gn_b[None, :],
      Wq2, bq2[None, :], Wk2, bk2[None, :], Wv2, bv2[None, :],
      Ws2, bs2[None, :], jnp.asarray(p2), jnp.asarray(p2k), jnp.asarray(p2v))

    acc2 = _edge2(src, dst, q2, kv2, ef, jnp.zeros((NN, 32), f32))

    sn2 = np.zeros((32, 10), np.float32)
    sn2[:10, :] = np.eye(10)
    sd2 = np.zeros((32, 10), np.float32)
    sd2[16:26, :] = np.eye(10)
    mh2 = np.full((10, 1), 0.1, np.float32)
    wa = (Wb2[0] + Wb2[2])[None, :]
    wb = (Wb2[1] - Wb2[2])[None, :]
    out = pl.pallas_call(
        _final_body,
        grid=(ngrid,),
        in_specs=[pl.BlockSpec((NB, 32), lambda i: (i, 0)),
                  pl.BlockSpec((NB, 32), lambda i, n=nblk: (i + n, 0)),
                  _row_spec(1),
                  _bcast_spec((32, 10)), _bcast_spec((32, 10)),
                  _bcast_spec((10, 1)), _bcast_spec((1, 1)),
                  _bcast_spec((1, 1))],
        out_specs=_row_spec(1),
        out_shape=jax.ShapeDtypeStruct((NN, 1), f32),
    )(acc2, acc2, xr2, jnp.asarray(sn2), jnp.asarray(sd2), jnp.asarray(mh2),
      wa, wb)
    return out
